# Initial kernel scaffold; baseline (speedup 1.0000x reference)
import jax
import jax.numpy as jnp
from jax.experimental import pallas as pl


def kernel(x, edge_index, c1_W, c1_b, mean_W, mean_b, W_ih, W_hh, b_ih, b_hh, lin_W, lin_b):
    raise NotImplementedError("write your pallas kernel here")


# speedup vs baseline: 23.3378x; 23.3378x over previous
import functools

import jax
import jax.numpy as jnp
from jax import lax
from jax.experimental import pallas as pl
from jax.experimental.pallas import tpu as pltpu
from jax.experimental.pallas import tpu_sc as plsc

N_CORES = 2
N_SUB = 16
NW = N_CORES * N_SUB
BATCH = 128
ROW_BLK = 1280


def _sc_mesh():
    return plsc.VectorSubcoreMesh(core_axis_name="c", subcore_axis_name="s")


def _make_deg_kernel(NP, K):
    RPS = NP // N_SUB

    @functools.partial(
        pl.kernel,
        out_type=jax.ShapeDtypeStruct((N_CORES, NP, 16), jnp.float32),
        mesh=_sc_mesh(),
        compiler_params=pltpu.CompilerParams(use_tc_tiling_on_sc=False),
        scratch_types=[
            pltpu.VMEM((K, BATCH), jnp.int32),
            pltpu.VMEM((BATCH, 16), jnp.float32),
            pltpu.VMEM_SHARED((NP, 16), jnp.float32),
        ],
    )
    def deg_kernel(dst_hbm, ones_hbm, zeros_hbm, out_hbm, dst_v, ones_v, acc):
        cid = lax.axis_index("c")
        sid = lax.axis_index("s")
        wid = cid * N_SUB + sid
        pltpu.sync_copy(dst_hbm.at[wid], dst_v)
        pltpu.sync_copy(ones_hbm, ones_v)
        pltpu.sync_copy(zeros_hbm.at[pl.ds(sid * RPS, RPS)],
                        acc.at[pl.ds(sid * RPS, RPS)])
        plsc.subcore_barrier()

        def body(j, carry):
            pltpu.sync_copy(ones_v, acc.at[dst_v.at[j]], add=True)
            return carry

        lax.fori_loop(0, K, body, 0)
        plsc.subcore_barrier()
        pltpu.sync_copy(acc.at[pl.ds(sid * RPS, RPS)],
                        out_hbm.at[cid, pl.ds(sid * RPS, RPS)])

    return deg_kernel


def _make_conv_kernel(NP, K, H):
    RPS = NP // N_SUB

    @functools.partial(
        pl.kernel,
        out_type=jax.ShapeDtypeStruct((N_CORES, NP, H), jnp.float32),
        mesh=_sc_mesh(),
        compiler_params=pltpu.CompilerParams(use_tc_tiling_on_sc=False),
        scratch_types=[
            pltpu.VMEM((K, BATCH), jnp.int32),
            pltpu.VMEM((K, BATCH), jnp.int32),
            pltpu.VMEM((BATCH, H), jnp.float32),
            pltpu.VMEM_SHARED((NP, H), jnp.float32),
            pltpu.SemaphoreType.DMA,
        ],
    )
    def conv_kernel(table_hbm, src_hbm, dst_hbm, zeros_hbm, out_hbm,
                    src_v, dst_v, rows_v, acc, sem):
        cid = lax.axis_index("c")
        sid = lax.axis_index("s")
        wid = cid * N_SUB + sid
        pltpu.sync_copy(src_hbm.at[wid], src_v)
        pltpu.sync_copy(dst_hbm.at[wid], dst_v)
        pltpu.sync_copy(zeros_hbm.at[pl.ds(sid * RPS, RPS)],
                        acc.at[pl.ds(sid * RPS, RPS)])
        plsc.subcore_barrier()

        def body(j, carry):
            pltpu.async_copy(table_hbm.at[src_v.at[j]], rows_v, sem).wait()
            pltpu.sync_copy(rows_v, acc.at[dst_v.at[j]], add=True)
            return carry

        lax.fori_loop(0, K, body, 0)
        plsc.subcore_barrier()
        pltpu.sync_copy(acc.at[pl.ds(sid * RPS, RPS)],
                        out_hbm.at[cid, pl.ds(sid * RPS, RPS)])

    return conv_kernel


def _tc1_body(x_ref, w_ref, degp_ref, hs_ref, dinv_ref):
    deg = 1.0 + degp_ref[0, :, 0:1] + degp_ref[1, :, 0:1]
    dinv = lax.rsqrt(deg)
    dinv32 = jnp.broadcast_to(dinv, hs_ref.shape)
    h = jnp.dot(x_ref[...], w_ref[...], preferred_element_type=jnp.float32)
    dinv_ref[...] = dinv32
    hs_ref[...] = h * dinv32


def _tc2_body(dinv_ref, p_ref, hs1_ref, b1_ref, w2_ref, hs2_ref):
    dinv = dinv_ref[...]
    h1 = jnp.maximum(
        dinv * (p_ref[0] + p_ref[1] + hs1_ref[...]) + b1_ref[...], 0.0)
    h2 = jnp.dot(h1, w2_ref[...], preferred_element_type=jnp.float32)
    hs2_ref[...] = h2 * dinv


def _tc3_body(T, H, dinv_ref, q_ref, hs2_ref, b2_ref, wih_ref, whh_ref,
              bih_ref, bhh_ref, linw_ref, linb_ref, out_ref):
    dinv = dinv_ref[...]
    z = dinv * (q_ref[0] + q_ref[1] + hs2_ref[...]) + b2_ref[...]
    xs = jnp.tanh(z)
    gi = jnp.dot(xs, wih_ref[...], preferred_element_type=jnp.float32)
    gi = gi + bih_ref[...]
    h = jnp.zeros_like(z)
    for t in range(T):
        gh = jnp.dot(h, whh_ref[...], preferred_element_type=jnp.float32)
        gh = gh + bhh_ref[...]
        r = jax.nn.sigmoid(gi[:, 0:H] + gh[:, 0:H])
        zg = jax.nn.sigmoid(gi[:, H:2 * H] + gh[:, H:2 * H])
        n = jnp.tanh(gi[:, 2 * H:3 * H] + r * gh[:, 2 * H:3 * H])
        h = (1.0 - zg) * n + zg * h
        out_ref[t] = (jnp.dot(h, linw_ref[...],
                              preferred_element_type=jnp.float32)
                      + linb_ref[...])


def kernel(x, edge_index, c1_W, c1_b, mean_W, mean_b, W_ih, W_hh, b_ih, b_hh,
           lin_W, lin_b):
    N, D = x.shape
    H = c1_W.shape[1]
    G = W_ih.shape[0]
    Z = lin_W.shape[0]
    T = 4
    E = edge_index.shape[1]

    NP = -(-(N + 1) // ROW_BLK) * ROW_BLK
    GRID = NP // ROW_BLK
    K = -(-E // (NW * BATCH))
    EPAD = NW * K * BATCH

    f32 = jnp.float32
    pad_idx = jnp.full((EPAD - E,), N, dtype=jnp.int32)
    src = jnp.concatenate([edge_index[0], pad_idx]).reshape(NW, K, BATCH)
    dst = jnp.concatenate([edge_index[1], pad_idx]).reshape(NW, K, BATCH)
    xp = jnp.pad(x, ((0, NP - N), (0, 0)))

    ones16 = jnp.ones((BATCH, 16), f32)
    zeros16 = jnp.zeros((NP, 16), f32)
    zerosH = jnp.zeros((NP, H), f32)

    deg_kernel = _make_deg_kernel(NP, K)
    conv_kernel = _make_conv_kernel(NP, K, H)

    degp = deg_kernel(dst, ones16, zeros16)

    row_spec = lambda w: pl.BlockSpec((ROW_BLK, w), lambda i: (i, 0))
    full_spec = lambda a, b: pl.BlockSpec((a, b), lambda i: (0, 0))
    part_spec = lambda w: pl.BlockSpec((N_CORES, ROW_BLK, w),
                                       lambda i: (0, i, 0))

    hs1, dinv32 = pl.pallas_call(
        _tc1_body,
        grid=(GRID,),
        in_specs=[row_spec(D), full_spec(D, H), part_spec(16)],
        out_specs=[row_spec(H), row_spec(H)],
        out_shape=[jax.ShapeDtypeStruct((NP, H), f32),
                   jax.ShapeDtypeStruct((NP, H), f32)],
    )(xp, c1_W, degp)

    p = conv_kernel(hs1, src, dst, zerosH)

    hs2 = pl.pallas_call(
        _tc2_body,
        grid=(GRID,),
        in_specs=[row_spec(H), part_spec(H), row_spec(H),
                  full_spec(1, H), full_spec(H, H)],
        out_specs=row_spec(H),
        out_shape=jax.ShapeDtypeStruct((NP, H), f32),
    )(dinv32, p, hs1, c1_b.reshape(1, H), mean_W)

    q = conv_kernel(hs2, src, dst, zerosH)

    outp = pl.pallas_call(
        functools.partial(_tc3_body, T, H),
        grid=(GRID,),
        in_specs=[row_spec(H), part_spec(H), row_spec(H), full_spec(1, H),
                  full_spec(H, G), full_spec(H, G), full_spec(1, G),
                  full_spec(1, G), full_spec(H, Z), full_spec(1, Z)],
        out_specs=pl.BlockSpec((T, ROW_BLK, Z), lambda i: (0, i, 0)),
        out_shape=jax.ShapeDtypeStruct((T, NP, Z), f32),
    )(dinv32, q, hs2, mean_b.reshape(1, H), W_ih.T, W_hh.T,
      b_ih.reshape(1, G), b_hh.reshape(1, G), lin_W.T, lin_b.reshape(1, Z))

    return outp[:, :N, :]


# speedup vs baseline: 23.6243x; 1.0123x over previous
import functools

import jax
import jax.numpy as jnp
from jax import lax
from jax.experimental import pallas as pl
from jax.experimental.pallas import tpu as pltpu
from jax.experimental.pallas import tpu_sc as plsc

N_CORES = 2
N_SUB = 16
NW = N_CORES * N_SUB
BATCH = 128
ROW_BLK = 1280


def _sc_mesh():
    return plsc.VectorSubcoreMesh(core_axis_name="c", subcore_axis_name="s")


def _make_deg_kernel(NP, K):
    RPS = NP // N_SUB

    @functools.partial(
        pl.kernel,
        out_type=jax.ShapeDtypeStruct((N_CORES, NP, 16), jnp.float32),
        mesh=_sc_mesh(),
        compiler_params=pltpu.CompilerParams(use_tc_tiling_on_sc=False),
        scratch_types=[
            pltpu.VMEM((K, BATCH), jnp.int32),
            pltpu.VMEM((BATCH, 16), jnp.float32),
            pltpu.VMEM_SHARED((NP, 16), jnp.float32),
        ],
    )
    def deg_kernel(dst_hbm, ones_hbm, zeros_hbm, out_hbm, dst_v, ones_v, acc):
        cid = lax.axis_index("c")
        sid = lax.axis_index("s")
        wid = cid * N_SUB + sid
        pltpu.sync_copy(dst_hbm.at[wid], dst_v)
        pltpu.sync_copy(ones_hbm, ones_v)
        pltpu.sync_copy(zeros_hbm.at[pl.ds(sid * RPS, RPS)],
                        acc.at[pl.ds(sid * RPS, RPS)])
        plsc.subcore_barrier()

        def body(j, carry):
            pltpu.sync_copy(ones_v, acc.at[dst_v.at[j]], add=True)
            return carry

        lax.fori_loop(0, K, body, 0)
        plsc.subcore_barrier()
        pltpu.sync_copy(acc.at[pl.ds(sid * RPS, RPS)],
                        out_hbm.at[cid, pl.ds(sid * RPS, RPS)])

    return deg_kernel


NBUF = 8


def _make_conv_kernel(NP, K, H):
    RPS = NP // N_SUB
    NG = K // NBUF

    @functools.partial(
        pl.kernel,
        out_type=jax.ShapeDtypeStruct((N_CORES, NP, H), jnp.float32),
        mesh=_sc_mesh(),
        compiler_params=pltpu.CompilerParams(use_tc_tiling_on_sc=False),
        scratch_types=[
            pltpu.VMEM((K, BATCH), jnp.int32),
            pltpu.VMEM((K, BATCH), jnp.int32),
            pltpu.VMEM_SHARED((NP, H), jnp.float32),
        ] + [pltpu.VMEM((BATCH, H), jnp.float32) for _ in range(NBUF)]
          + [pltpu.SemaphoreType.DMA for _ in range(NBUF)]
          + [pltpu.SemaphoreType.DMA],
    )
    def conv_kernel(table_hbm, src_hbm, dst_hbm, zeros_hbm, out_hbm,
                    src_v, dst_v, acc, *bufs_and_sems):
        rows = bufs_and_sems[:NBUF]
        gsem = bufs_and_sems[NBUF:2 * NBUF]
        ssem = bufs_and_sems[2 * NBUF]
        cid = lax.axis_index("c")
        sid = lax.axis_index("s")
        wid = cid * N_SUB + sid
        pltpu.sync_copy(src_hbm.at[wid], src_v)
        pltpu.sync_copy(dst_hbm.at[wid], dst_v)
        pltpu.sync_copy(zeros_hbm.at[pl.ds(sid * RPS, RPS)],
                        acc.at[pl.ds(sid * RPS, RPS)])
        plsc.subcore_barrier()

        for b in range(NBUF):
            pltpu.async_copy(table_hbm.at[src_v.at[b]], rows[b], gsem[b])

        def group(g, carry):
            for b in range(NBUF):
                j = g * NBUF + b
                pltpu.make_async_copy(
                    table_hbm.at[src_v.at[j]], rows[b], gsem[b]).wait()
                pltpu.async_copy(rows[b], acc.at[dst_v.at[j]], ssem,
                                 add=True)
            for b in range(NBUF):
                j = g * NBUF + b
                pltpu.make_async_copy(rows[b], acc.at[dst_v.at[j]],
                                      ssem).wait()
            jn = jnp.minimum((g + 1) * NBUF, K - NBUF)
            for b in range(NBUF):
                pltpu.async_copy(table_hbm.at[src_v.at[jn + b]], rows[b],
                                 gsem[b])
            return carry

        lax.fori_loop(0, NG, group, 0)
        for b in range(NBUF):
            pltpu.make_async_copy(
                table_hbm.at[src_v.at[K - NBUF + b]], rows[b],
                gsem[b]).wait()
        plsc.subcore_barrier()
        pltpu.sync_copy(acc.at[pl.ds(sid * RPS, RPS)],
                        out_hbm.at[cid, pl.ds(sid * RPS, RPS)])

    return conv_kernel


def _tc1_body(x_ref, w_ref, degp_ref, hs_ref, dinv_ref):
    deg = 1.0 + degp_ref[0, :, 0:1] + degp_ref[1, :, 0:1]
    dinv = lax.rsqrt(deg)
    dinv32 = jnp.broadcast_to(dinv, hs_ref.shape)
    h = jnp.dot(x_ref[...], w_ref[...], preferred_element_type=jnp.float32)
    dinv_ref[...] = dinv32
    hs_ref[...] = h * dinv32


def _tc2_body(dinv_ref, p_ref, hs1_ref, b1_ref, w2_ref, hs2_ref):
    dinv = dinv_ref[...]
    h1 = jnp.maximum(
        dinv * (p_ref[0] + p_ref[1] + hs1_ref[...]) + b1_ref[...], 0.0)
    h2 = jnp.dot(h1, w2_ref[...], preferred_element_type=jnp.float32)
    hs2_ref[...] = h2 * dinv


def _tc3_body(T, H, dinv_ref, q_ref, hs2_ref, b2_ref, wih_ref, whh_ref,
              bih_ref, bhh_ref, linw_ref, linb_ref, out_ref):
    dinv = dinv_ref[...]
    z = dinv * (q_ref[0] + q_ref[1] + hs2_ref[...]) + b2_ref[...]
    xs = jnp.tanh(z)
    gi = jnp.dot(xs, wih_ref[...], preferred_element_type=jnp.float32)
    gi = gi + bih_ref[...]
    h = jnp.zeros_like(z)
    for t in range(T):
        gh = jnp.dot(h, whh_ref[...], preferred_element_type=jnp.float32)
        gh = gh + bhh_ref[...]
        r = jax.nn.sigmoid(gi[:, 0:H] + gh[:, 0:H])
        zg = jax.nn.sigmoid(gi[:, H:2 * H] + gh[:, H:2 * H])
        n = jnp.tanh(gi[:, 2 * H:3 * H] + r * gh[:, 2 * H:3 * H])
        h = (1.0 - zg) * n + zg * h
        out_ref[t] = (jnp.dot(h, linw_ref[...],
                              preferred_element_type=jnp.float32)
                      + linb_ref[...])


def kernel(x, edge_index, c1_W, c1_b, mean_W, mean_b, W_ih, W_hh, b_ih, b_hh,
           lin_W, lin_b):
    N, D = x.shape
    H = c1_W.shape[1]
    G = W_ih.shape[0]
    Z = lin_W.shape[0]
    T = 4
    E = edge_index.shape[1]

    NP = -(-(N + 1) // ROW_BLK) * ROW_BLK
    GRID = NP // ROW_BLK
    K = -(-E // (NW * BATCH))
    K = -(-K // NBUF) * NBUF
    EPAD = NW * K * BATCH

    f32 = jnp.float32
    pad_idx = jnp.full((EPAD - E,), N, dtype=jnp.int32)
    src = jnp.concatenate([edge_index[0], pad_idx]).reshape(NW, K, BATCH)
    dst = jnp.concatenate([edge_index[1], pad_idx]).reshape(NW, K, BATCH)
    xp = jnp.pad(x, ((0, NP - N), (0, 0)))

    ones16 = jnp.ones((BATCH, 16), f32)
    zeros16 = jnp.zeros((NP, 16), f32)
    zerosH = jnp.zeros((NP, H), f32)

    deg_kernel = _make_deg_kernel(NP, K)
    conv_kernel = _make_conv_kernel(NP, K, H)

    degp = deg_kernel(dst, ones16, zeros16)

    row_spec = lambda w: pl.BlockSpec((ROW_BLK, w), lambda i: (i, 0))
    full_spec = lambda a, b: pl.BlockSpec((a, b), lambda i: (0, 0))
    part_spec = lambda w: pl.BlockSpec((N_CORES, ROW_BLK, w),
                                       lambda i: (0, i, 0))

    hs1, dinv32 = pl.pallas_call(
        _tc1_body,
        grid=(GRID,),
        in_specs=[row_spec(D), full_spec(D, H), part_spec(16)],
        out_specs=[row_spec(H), row_spec(H)],
        out_shape=[jax.ShapeDtypeStruct((NP, H), f32),
                   jax.ShapeDtypeStruct((NP, H), f32)],
    )(xp, c1_W, degp)

    p = conv_kernel(hs1, src, dst, zerosH)

    hs2 = pl.pallas_call(
        _tc2_body,
        grid=(GRID,),
        in_specs=[row_spec(H), part_spec(H), row_spec(H),
                  full_spec(1, H), full_spec(H, H)],
        out_specs=row_spec(H),
        out_shape=jax.ShapeDtypeStruct((NP, H), f32),
    )(dinv32, p, hs1, c1_b.reshape(1, H), mean_W)

    q = conv_kernel(hs2, src, dst, zerosH)

    outp = pl.pallas_call(
        functools.partial(_tc3_body, T, H),
        grid=(GRID,),
        in_specs=[row_spec(H), part_spec(H), row_spec(H), full_spec(1, H),
                  full_spec(H, G), full_spec(H, G), full_spec(1, G),
                  full_spec(1, G), full_spec(H, Z), full_spec(1, Z)],
        out_specs=pl.BlockSpec((T, ROW_BLK, Z), lambda i: (0, i, 0)),
        out_shape=jax.ShapeDtypeStruct((T, NP, Z), f32),
    )(dinv32, q, hs2, mean_b.reshape(1, H), W_ih.T, W_hh.T,
      b_ih.reshape(1, G), b_hh.reshape(1, G), lin_W.T, lin_b.reshape(1, Z))

    return outp[:, :N, :]


# speedup vs baseline: 40.3561x; 1.7082x over previous
import functools

import jax
import jax.numpy as jnp
from jax import lax
from jax.experimental import pallas as pl
from jax.experimental.pallas import tpu as pltpu
from jax.experimental.pallas import tpu_sc as plsc

N_CORES = 2
N_SUB = 16
NW = N_CORES * N_SUB
BATCH = 128
ROW_BLK = 1280


def _sc_mesh():
    return plsc.VectorSubcoreMesh(core_axis_name="c", subcore_axis_name="s")


def _make_deg_kernel(NP, K):
    RPS = NP // N_SUB

    @functools.partial(
        pl.kernel,
        out_type=jax.ShapeDtypeStruct((N_CORES, NP, 16), jnp.float32),
        mesh=_sc_mesh(),
        compiler_params=pltpu.CompilerParams(use_tc_tiling_on_sc=False),
        scratch_types=[
            pltpu.VMEM((K, BATCH), jnp.int32),
            pltpu.VMEM((BATCH, 16), jnp.float32),
            pltpu.VMEM_SHARED((NP, 16), jnp.float32),
        ],
    )
    def deg_kernel(dst_hbm, ones_hbm, zeros_hbm, out_hbm, dst_v, ones_v, acc):
        cid = lax.axis_index("c")
        sid = lax.axis_index("s")
        wid = cid * N_SUB + sid
        pltpu.sync_copy(dst_hbm.at[wid], dst_v)
        pltpu.sync_copy(ones_hbm, ones_v)
        pltpu.sync_copy(zeros_hbm.at[pl.ds(sid * RPS, RPS)],
                        acc.at[pl.ds(sid * RPS, RPS)])
        plsc.subcore_barrier()

        def body(j, carry):
            pltpu.sync_copy(ones_v, acc.at[dst_v.at[j]], add=True)
            return carry

        lax.fori_loop(0, K, body, 0)
        plsc.subcore_barrier()
        pltpu.sync_copy(acc.at[pl.ds(sid * RPS, RPS)],
                        out_hbm.at[cid, pl.ds(sid * RPS, RPS)])

    return deg_kernel


NBUF = 8


def _make_conv_kernel(NP, K, H):
    RPS = NP // N_SUB
    NG = K // NBUF

    @functools.partial(
        pl.kernel,
        out_type=jax.ShapeDtypeStruct((N_CORES, NP, H), jnp.float32),
        mesh=_sc_mesh(),
        compiler_params=pltpu.CompilerParams(use_tc_tiling_on_sc=False),
        scratch_types=[
            pltpu.VMEM((K, BATCH), jnp.int32),
            pltpu.VMEM((K, BATCH), jnp.int32),
            pltpu.VMEM_SHARED((NP, H), jnp.float32),
        ] + [pltpu.VMEM((BATCH, H), jnp.float32) for _ in range(NBUF)]
          + [pltpu.SemaphoreType.DMA for _ in range(NBUF)]
          + [pltpu.SemaphoreType.DMA],
    )
    def conv_kernel(table_hbm, src_hbm, dst_hbm, zeros_hbm, out_hbm,
                    src_v, dst_v, acc, *bufs_and_sems):
        rows = bufs_and_sems[:NBUF]
        gsem = bufs_and_sems[NBUF:2 * NBUF]
        ssem = bufs_and_sems[2 * NBUF]
        cid = lax.axis_index("c")
        sid = lax.axis_index("s")
        wid = cid * N_SUB + sid
        pltpu.sync_copy(src_hbm.at[wid], src_v)
        pltpu.sync_copy(dst_hbm.at[wid], dst_v)
        pltpu.sync_copy(zeros_hbm.at[pl.ds(sid * RPS, RPS)],
                        acc.at[pl.ds(sid * RPS, RPS)])
        plsc.subcore_barrier()

        for b in range(NBUF):
            pltpu.async_copy(table_hbm.at[src_v.at[b]], rows[b], gsem[b])

        def group(g, carry):
            for b in range(NBUF):
                j = g * NBUF + b
                pltpu.make_async_copy(
                    table_hbm.at[src_v.at[j]], rows[b], gsem[b]).wait()
                pltpu.async_copy(rows[b], acc.at[dst_v.at[j]], ssem,
                                 add=True)
            for b in range(NBUF):
                j = g * NBUF + b
                pltpu.make_async_copy(rows[b], acc.at[dst_v.at[j]],
                                      ssem).wait()
            jn = jnp.minimum((g + 1) * NBUF, K - NBUF)
            for b in range(NBUF):
                pltpu.async_copy(table_hbm.at[src_v.at[jn + b]], rows[b],
                                 gsem[b])
            return carry

        lax.fori_loop(0, NG, group, 0)
        for b in range(NBUF):
            pltpu.make_async_copy(
                table_hbm.at[src_v.at[K - NBUF + b]], rows[b],
                gsem[b]).wait()
        plsc.subcore_barrier()
        pltpu.sync_copy(acc.at[pl.ds(sid * RPS, RPS)],
                        out_hbm.at[cid, pl.ds(sid * RPS, RPS)])

    return conv_kernel


def _tc1_body(x_ref, w_ref, degp_ref, hs_ref, dinv_ref):
    deg = 1.0 + degp_ref[0, :, 0:1] + degp_ref[1, :, 0:1]
    dinv = lax.rsqrt(deg)
    dinv32 = jnp.broadcast_to(dinv, hs_ref.shape)
    h = jnp.dot(x_ref[...], w_ref[...], preferred_element_type=jnp.float32)
    dinv_ref[...] = dinv32
    hs_ref[...] = h * dinv32


def _tc2_body(dinv_ref, p_ref, hs1_ref, b1_ref, w2_ref, hs2_ref):
    dinv = dinv_ref[...]
    h1 = jnp.maximum(
        dinv * (p_ref[0] + p_ref[1] + hs1_ref[...]) + b1_ref[...], 0.0)
    h2 = jnp.dot(h1, w2_ref[...], preferred_element_type=jnp.float32)
    hs2_ref[...] = h2 * dinv


def _tc3_body(T, H, dinv_ref, q_ref, hs2_ref, b2_ref, wih_ref, whh_ref,
              bih_ref, bhh_ref, linw_ref, linb_ref, out_ref):
    dinv = dinv_ref[...]
    z = dinv * (q_ref[0] + q_ref[1] + hs2_ref[...]) + b2_ref[...]
    xs = jnp.tanh(z)
    gi = jnp.dot(xs, wih_ref[...], preferred_element_type=jnp.float32)
    gi = gi + bih_ref[...]
    h = jnp.zeros_like(z)
    for t in range(T):
        gh = jnp.dot(h, whh_ref[...], preferred_element_type=jnp.float32)
        gh = gh + bhh_ref[...]
        r = jax.nn.sigmoid(gi[:, 0:H] + gh[:, 0:H])
        zg = jax.nn.sigmoid(gi[:, H:2 * H] + gh[:, H:2 * H])
        n = jnp.tanh(gi[:, 2 * H:3 * H] + r * gh[:, 2 * H:3 * H])
        h = (1.0 - zg) * n + zg * h
        out_ref[t] = (jnp.dot(h, linw_ref[...],
                              preferred_element_type=jnp.float32)
                      + linb_ref[...])


def kernel(x, edge_index, c1_W, c1_b, mean_W, mean_b, W_ih, W_hh, b_ih, b_hh,
           lin_W, lin_b):
    N, D = x.shape
    H = c1_W.shape[1]
    G = W_ih.shape[0]
    Z = lin_W.shape[0]
    T = 4
    E = edge_index.shape[1]

    NP = -(-(N + 1) // ROW_BLK) * ROW_BLK
    GRID = NP // ROW_BLK
    K = -(-E // (NW * BATCH))
    K = -(-K // NBUF) * NBUF
    EPAD = NW * K * BATCH

    f32 = jnp.float32
    pad_idx = N + jnp.arange(EPAD - E, dtype=jnp.int32) % (NP - N)
    src = jnp.concatenate([edge_index[0], pad_idx]).reshape(NW, K, BATCH)
    dst = jnp.concatenate([edge_index[1], pad_idx]).reshape(NW, K, BATCH)
    xp = jnp.pad(x, ((0, NP - N), (0, 0)))

    ones16 = jnp.ones((BATCH, 16), f32)
    zeros16 = jnp.zeros((NP, 16), f32)
    zerosH = jnp.zeros((NP, H), f32)

    deg_kernel = _make_deg_kernel(NP, K)
    conv_kernel = _make_conv_kernel(NP, K, H)

    degp = deg_kernel(dst, ones16, zeros16)

    row_spec = lambda w: pl.BlockSpec((ROW_BLK, w), lambda i: (i, 0))
    full_spec = lambda a, b: pl.BlockSpec((a, b), lambda i: (0, 0))
    part_spec = lambda w: pl.BlockSpec((N_CORES, ROW_BLK, w),
                                       lambda i: (0, i, 0))

    hs1, dinv32 = pl.pallas_call(
        _tc1_body,
        grid=(GRID,),
        in_specs=[row_spec(D), full_spec(D, H), part_spec(16)],
        out_specs=[row_spec(H), row_spec(H)],
        out_shape=[jax.ShapeDtypeStruct((NP, H), f32),
                   jax.ShapeDtypeStruct((NP, H), f32)],
    )(xp, c1_W, degp)

    p = conv_kernel(hs1, src, dst, zerosH)

    hs2 = pl.pallas_call(
        _tc2_body,
        grid=(GRID,),
        in_specs=[row_spec(H), part_spec(H), row_spec(H),
                  full_spec(1, H), full_spec(H, H)],
        out_specs=row_spec(H),
        out_shape=jax.ShapeDtypeStruct((NP, H), f32),
    )(dinv32, p, hs1, c1_b.reshape(1, H), mean_W)

    q = conv_kernel(hs2, src, dst, zerosH)

    R3 = next((r for r in (1280, 1000, 800, 500, 250, 200, 8)
               if N % r == 0), None)
    rows3 = N if R3 is None else N
    if R3 is None:
        R3, rows3 = ROW_BLK, NP
    row3_spec = lambda w: pl.BlockSpec((R3, w), lambda i: (i, 0))
    full3_spec = lambda a, b: pl.BlockSpec((a, b), lambda i: (0, 0))
    outp = pl.pallas_call(
        functools.partial(_tc3_body, T, H),
        grid=(rows3 // R3,),
        in_specs=[row3_spec(H),
                  pl.BlockSpec((N_CORES, R3, H), lambda i: (0, i, 0)),
                  row3_spec(H), full3_spec(1, H),
                  full3_spec(H, G), full3_spec(H, G), full3_spec(1, G),
                  full3_spec(1, G), full3_spec(H, Z), full3_spec(1, Z)],
        out_specs=pl.BlockSpec((T, R3, Z), lambda i: (0, i, 0)),
        out_shape=jax.ShapeDtypeStruct((T, rows3, Z), f32),
    )(dinv32, q, hs2, mean_b.reshape(1, H), W_ih.T, W_hh.T,
      b_ih.reshape(1, G), b_hh.reshape(1, G), lin_W.T, lin_b.reshape(1, Z))

    return outp[:, :N, :] if rows3 != N else outp


# speedup vs baseline: 40.3650x; 1.0002x over previous
import functools

import numpy as np
import jax
import jax.numpy as jnp
from jax import lax
from jax.experimental import pallas as pl
from jax.experimental.pallas import tpu as pltpu
from jax.experimental.pallas import tpu_sc as plsc

N_CORES = 2
N_SUB = 16
NW = N_CORES * N_SUB
BATCH = 128
ROW_BLK = 1280


def _sc_mesh():
    return plsc.VectorSubcoreMesh(core_axis_name="c", subcore_axis_name="s")


def _make_deg_kernel(NP, K):
    RPS = NP // N_SUB

    @functools.partial(
        pl.kernel,
        out_type=jax.ShapeDtypeStruct((N_CORES, NP, 16), jnp.float32),
        mesh=_sc_mesh(),
        compiler_params=pltpu.CompilerParams(use_tc_tiling_on_sc=False),
        scratch_types=[
            pltpu.VMEM((K, BATCH), jnp.int32),
            pltpu.VMEM((BATCH, 16), jnp.float32),
            pltpu.VMEM_SHARED((NP, 16), jnp.float32),
        ],
    )
    def deg_kernel(dst_hbm, ones_hbm, zeros_hbm, out_hbm, dst_v, ones_v, acc):
        cid = lax.axis_index("c")
        sid = lax.axis_index("s")
        wid = cid * N_SUB + sid
        pltpu.sync_copy(dst_hbm.at[wid], dst_v)
        pltpu.sync_copy(ones_hbm, ones_v)
        pltpu.sync_copy(zeros_hbm.at[pl.ds(sid * RPS, RPS)],
                        acc.at[pl.ds(sid * RPS, RPS)])
        plsc.subcore_barrier()

        def body(j, carry):
            pltpu.sync_copy(ones_v, acc.at[dst_v.at[j]], add=True)
            return carry

        lax.fori_loop(0, K, body, 0)
        plsc.subcore_barrier()
        pltpu.sync_copy(acc.at[pl.ds(sid * RPS, RPS)],
                        out_hbm.at[cid, pl.ds(sid * RPS, RPS)])

    return deg_kernel


NBUF = 16


def _make_conv_kernel(NP, K, H):
    RPS = NP // N_SUB
    NG = K // NBUF

    @functools.partial(
        pl.kernel,
        out_type=jax.ShapeDtypeStruct((N_CORES, NP, H), jnp.float32),
        mesh=_sc_mesh(),
        compiler_params=pltpu.CompilerParams(use_tc_tiling_on_sc=False),
        scratch_types=[
            pltpu.VMEM((K, BATCH), jnp.int32),
            pltpu.VMEM((K, BATCH), jnp.int32),
            pltpu.VMEM_SHARED((NP, H), jnp.float32),
        ] + [pltpu.VMEM((BATCH, H), jnp.float32) for _ in range(NBUF)]
          + [pltpu.SemaphoreType.DMA for _ in range(NBUF)]
          + [pltpu.SemaphoreType.DMA],
    )
    def conv_kernel(table_hbm, src_hbm, dst_hbm, zeros_hbm, out_hbm,
                    src_v, dst_v, acc, *bufs_and_sems):
        rows = bufs_and_sems[:NBUF]
        gsem = bufs_and_sems[NBUF:2 * NBUF]
        ssem = bufs_and_sems[2 * NBUF]
        cid = lax.axis_index("c")
        sid = lax.axis_index("s")
        wid = cid * N_SUB + sid
        pltpu.sync_copy(src_hbm.at[wid], src_v)
        pltpu.sync_copy(dst_hbm.at[wid], dst_v)
        pltpu.sync_copy(zeros_hbm.at[pl.ds(sid * RPS, RPS)],
                        acc.at[pl.ds(sid * RPS, RPS)])
        plsc.subcore_barrier()

        for b in range(NBUF):
            pltpu.async_copy(table_hbm.at[src_v.at[b]], rows[b], gsem[b])

        def group(g, carry):
            for b in range(NBUF):
                j = g * NBUF + b
                pltpu.make_async_copy(
                    table_hbm.at[src_v.at[j]], rows[b], gsem[b]).wait()
                pltpu.async_copy(rows[b], acc.at[dst_v.at[j]], ssem,
                                 add=True)
            for b in range(NBUF):
                j = g * NBUF + b
                pltpu.make_async_copy(rows[b], acc.at[dst_v.at[j]],
                                      ssem).wait()
            jn = jnp.minimum((g + 1) * NBUF, K - NBUF)
            for b in range(NBUF):
                pltpu.async_copy(table_hbm.at[src_v.at[jn + b]], rows[b],
                                 gsem[b])
            return carry

        lax.fori_loop(0, NG, group, 0)
        for b in range(NBUF):
            pltpu.make_async_copy(
                table_hbm.at[src_v.at[K - NBUF + b]], rows[b],
                gsem[b]).wait()
        plsc.subcore_barrier()
        pltpu.sync_copy(acc.at[pl.ds(sid * RPS, RPS)],
                        out_hbm.at[cid, pl.ds(sid * RPS, RPS)])

    return conv_kernel


def _tc1_body(x_ref, w_ref, degp_ref, hs_ref, dinv_ref):
    deg = 1.0 + degp_ref[0, :, 0:1] + degp_ref[1, :, 0:1]
    dinv = lax.rsqrt(deg)
    dinv32 = jnp.broadcast_to(dinv, hs_ref.shape)
    h = jnp.dot(x_ref[...], w_ref[...], preferred_element_type=jnp.float32)
    dinv_ref[...] = dinv32
    hs_ref[...] = h * dinv32


def _tc2_body(dinv_ref, p_ref, hs1_ref, b1_ref, w2_ref, hs2_ref):
    dinv = dinv_ref[...]
    h1 = jnp.maximum(
        dinv * (p_ref[0] + p_ref[1] + hs1_ref[...]) + b1_ref[...], 0.0)
    h2 = jnp.dot(h1, w2_ref[...], preferred_element_type=jnp.float32)
    hs2_ref[...] = h2 * dinv


def _tc3_body(T, H, dinv_ref, q_ref, hs2_ref, b2_ref, wih_ref, wcat_ref,
              bih_ref, bcat_ref, out_ref):
    dinv = dinv_ref[...]
    z = dinv * (q_ref[0] + q_ref[1] + hs2_ref[...]) + b2_ref[...]
    xs = jnp.tanh(z)
    gi = jnp.dot(xs, wih_ref[...], preferred_element_type=jnp.float32)
    gi = gi + bih_ref[...]
    h = jnp.zeros_like(z)
    for t in range(T):
        gc = jnp.dot(h, wcat_ref[...], preferred_element_type=jnp.float32)
        gc = gc + bcat_ref[...]
        if t > 0:
            out_ref[t - 1] = gc[:, 3 * H:]
        r = jax.nn.sigmoid(gi[:, 0:H] + gc[:, 0:H])
        zg = jax.nn.sigmoid(gi[:, H:2 * H] + gc[:, H:2 * H])
        n = jnp.tanh(gi[:, 2 * H:3 * H] + r * gc[:, 2 * H:3 * H])
        h = (1.0 - zg) * n + zg * h
    out_ref[T - 1] = (jnp.dot(h, wcat_ref[:, 3 * H:],
                              preferred_element_type=jnp.float32)
                      + bcat_ref[:, 3 * H:])


def kernel(x, edge_index, c1_W, c1_b, mean_W, mean_b, W_ih, W_hh, b_ih, b_hh,
           lin_W, lin_b):
    N, D = x.shape
    H = c1_W.shape[1]
    G = W_ih.shape[0]
    Z = lin_W.shape[0]
    T = 4
    E = edge_index.shape[1]

    NP = -(-(N + 1) // ROW_BLK) * ROW_BLK
    GRID = NP // ROW_BLK
    K = -(-E // (NW * BATCH))
    K = -(-K // NBUF) * NBUF
    EPAD = NW * K * BATCH

    f32 = jnp.float32
    pad_idx = jnp.asarray(
        N + np.arange(EPAD - E, dtype=np.int32) % (NP - N))
    src = jnp.concatenate([edge_index[0], pad_idx]).reshape(NW, K, BATCH)
    dst = jnp.concatenate([edge_index[1], pad_idx]).reshape(NW, K, BATCH)
    xp = jnp.pad(x, ((0, NP - N), (0, 0)))

    ones16 = jnp.ones((BATCH, 16), f32)
    zeros16 = jnp.zeros((NP, 16), f32)
    zerosH = jnp.zeros((NP, H), f32)

    deg_kernel = _make_deg_kernel(NP, K)
    conv_kernel = _make_conv_kernel(NP, K, H)

    degp = deg_kernel(dst, ones16, zeros16)

    row_spec = lambda w: pl.BlockSpec((ROW_BLK, w), lambda i: (i, 0))
    full_spec = lambda a, b: pl.BlockSpec((a, b), lambda i: (0, 0))
    part_spec = lambda w: pl.BlockSpec((N_CORES, ROW_BLK, w),
                                       lambda i: (0, i, 0))

    hs1, dinv32 = pl.pallas_call(
        _tc1_body,
        grid=(GRID,),
        in_specs=[row_spec(D), full_spec(D, H), part_spec(16)],
        out_specs=[row_spec(H), row_spec(H)],
        out_shape=[jax.ShapeDtypeStruct((NP, H), f32),
                   jax.ShapeDtypeStruct((NP, H), f32)],
    )(xp, c1_W, degp)

    p = conv_kernel(hs1, src, dst, zerosH)

    hs2 = pl.pallas_call(
        _tc2_body,
        grid=(GRID,),
        in_specs=[row_spec(H), part_spec(H), row_spec(H),
                  full_spec(1, H), full_spec(H, H)],
        out_specs=row_spec(H),
        out_shape=jax.ShapeDtypeStruct((NP, H), f32),
    )(dinv32, p, hs1, c1_b.reshape(1, H), mean_W)

    q = conv_kernel(hs2, src, dst, zerosH)

    R3 = next((r for r in (1280, 1000, 800, 500, 250, 200, 8)
               if N % r == 0), None)
    rows3 = N if R3 is None else N
    if R3 is None:
        R3, rows3 = ROW_BLK, NP
    row3_spec = lambda w: pl.BlockSpec((R3, w), lambda i: (i, 0))
    full3_spec = lambda a, b: pl.BlockSpec((a, b), lambda i: (0, 0))
    outp = pl.pallas_call(
        functools.partial(_tc3_body, T, H),
        grid=(rows3 // R3,),
        in_specs=[row3_spec(H),
                  pl.BlockSpec((N_CORES, R3, H), lambda i: (0, i, 0)),
                  row3_spec(H), full3_spec(1, H),
                  full3_spec(H, G), full3_spec(H, G + Z),
                  full3_spec(1, G), full3_spec(1, G + Z)],
        out_specs=pl.BlockSpec((T, R3, Z), lambda i: (0, i, 0)),
        out_shape=jax.ShapeDtypeStruct((T, rows3, Z), f32),
    )(dinv32, q, hs2, mean_b.reshape(1, H), W_ih.T,
      jnp.concatenate([W_hh.T, lin_W.T], axis=1),
      b_ih.reshape(1, G),
      jnp.concatenate([b_hh, lin_b]).reshape(1, G + Z))

    return outp[:, :N, :] if rows3 != N else outp


# speedup vs baseline: 46.3238x; 1.1476x over previous
import functools

import numpy as np
import jax
import jax.numpy as jnp
from jax import lax
from jax.experimental import pallas as pl
from jax.experimental.pallas import tpu as pltpu
from jax.experimental.pallas import tpu_sc as plsc

N_CORES = 2
N_SUB = 16
NW = N_CORES * N_SUB
BATCH = 128
ROW_BLK = 1280


def _sc_mesh():
    return plsc.VectorSubcoreMesh(core_axis_name="c", subcore_axis_name="s")


def _make_deg_kernel(NP, K, H):
    RPS = NP // N_SUB

    @functools.partial(
        pl.kernel,
        out_type=jax.ShapeDtypeStruct((N_CORES, NP, H), jnp.float32),
        mesh=_sc_mesh(),
        compiler_params=pltpu.CompilerParams(use_tc_tiling_on_sc=False),
        scratch_types=[
            pltpu.VMEM((K, BATCH), jnp.int32),
            pltpu.VMEM((BATCH, H), jnp.float32),
            pltpu.VMEM_SHARED((NP, H), jnp.float32),
        ],
    )
    def deg_kernel(dst_hbm, ones_hbm, zeros_hbm, out_hbm, dst_v, ones_v, acc):
        cid = lax.axis_index("c")
        sid = lax.axis_index("s")
        wid = cid * N_SUB + sid
        pltpu.sync_copy(dst_hbm.at[wid], dst_v)
        pltpu.sync_copy(ones_hbm, ones_v)
        pltpu.sync_copy(zeros_hbm.at[pl.ds(sid * RPS, RPS)],
                        acc.at[pl.ds(sid * RPS, RPS)])
        plsc.subcore_barrier()

        def body(j, carry):
            pltpu.sync_copy(ones_v, acc.at[dst_v.at[j]], add=True)
            return carry

        lax.fori_loop(0, K, body, 0)
        plsc.subcore_barrier()
        pltpu.sync_copy(acc.at[pl.ds(sid * RPS, RPS)],
                        out_hbm.at[cid, pl.ds(sid * RPS, RPS)])

    return deg_kernel


NBUF = 16


def _make_conv_kernel(NP, K, H):
    RPS = NP // N_SUB
    NG = K // NBUF

    @functools.partial(
        pl.kernel,
        out_type=jax.ShapeDtypeStruct((N_CORES, NP, H), jnp.float32),
        mesh=_sc_mesh(),
        compiler_params=pltpu.CompilerParams(use_tc_tiling_on_sc=False),
        scratch_types=[
            pltpu.VMEM((K, BATCH), jnp.int32),
            pltpu.VMEM((K, BATCH), jnp.int32),
            pltpu.VMEM_SHARED((NP, H), jnp.float32),
        ] + [pltpu.VMEM((BATCH, H), jnp.float32) for _ in range(NBUF)]
          + [pltpu.SemaphoreType.DMA for _ in range(NBUF)]
          + [pltpu.SemaphoreType.DMA],
    )
    def conv_kernel(table_hbm, src_hbm, dst_hbm, zeros_hbm, out_hbm,
                    src_v, dst_v, acc, *bufs_and_sems):
        rows = bufs_and_sems[:NBUF]
        gsem = bufs_and_sems[NBUF:2 * NBUF]
        ssem = bufs_and_sems[2 * NBUF]
        cid = lax.axis_index("c")
        sid = lax.axis_index("s")
        wid = cid * N_SUB + sid
        pltpu.sync_copy(src_hbm.at[wid], src_v)
        pltpu.sync_copy(dst_hbm.at[wid], dst_v)
        pltpu.sync_copy(zeros_hbm.at[pl.ds(sid * RPS, RPS)],
                        acc.at[pl.ds(sid * RPS, RPS)])
        plsc.subcore_barrier()

        for b in range(NBUF):
            pltpu.async_copy(table_hbm.at[src_v.at[b]], rows[b], gsem[b])

        def group(g, carry):
            for b in range(NBUF):
                j = g * NBUF + b
                pltpu.make_async_copy(
                    table_hbm.at[src_v.at[j]], rows[b], gsem[b]).wait()
                pltpu.async_copy(rows[b], acc.at[dst_v.at[j]], ssem,
                                 add=True)
            for b in range(NBUF):
                j = g * NBUF + b
                pltpu.make_async_copy(rows[b], acc.at[dst_v.at[j]],
                                      ssem).wait()
            jn = jnp.minimum((g + 1) * NBUF, K - NBUF)
            for b in range(NBUF):
                pltpu.async_copy(table_hbm.at[src_v.at[jn + b]], rows[b],
                                 gsem[b])
            return carry

        lax.fori_loop(0, NG, group, 0)
        for b in range(NBUF):
            pltpu.make_async_copy(
                table_hbm.at[src_v.at[K - NBUF + b]], rows[b],
                gsem[b]).wait()
        plsc.subcore_barrier()
        pltpu.sync_copy(acc.at[pl.ds(sid * RPS, RPS)],
                        out_hbm.at[cid, pl.ds(sid * RPS, RPS)])

    return conv_kernel


def _dot(a, b):
    return jnp.dot(a, b, preferred_element_type=jnp.float32)


def _tc1_body(x4_ref, w4_ref, degp_ref, hs_ref, dinv_ref):
    dinv = lax.rsqrt(1.0 + degp_ref[0] + degp_ref[1])
    h = _dot(x4_ref[...], w4_ref[...])
    dinv_ref[...] = dinv
    hs_ref[...] = h * dinv


def _tc2_body(dinv_ref, p_ref, hs1_ref, b1_ref, w4_ref, hs2_ref):
    dinv = dinv_ref[...]
    t = dinv * (p_ref[0] + p_ref[1] + hs1_ref[...]) + b1_ref[...]
    h1 = jnp.maximum(t, 0.0)
    hs2_ref[...] = _dot(h1, w4_ref[...]) * dinv


def _tc3_body(T, dinv_ref, q_ref, hs2_ref, b2_ref, wir_ref, wiz_ref,
              win_ref, whr_ref, whz_ref, whn_ref, lin4_ref, bir_ref,
              biz_ref, bin_ref, bhr_ref, bhz_ref, bhn_ref, blin_ref,
              out_ref):
    dinv = dinv_ref[...]
    z = dinv * (q_ref[0] + q_ref[1] + hs2_ref[...]) + b2_ref[...]
    xs = jnp.tanh(z)
    gir = _dot(xs, wir_ref[...]) + bir_ref[...]
    giz = _dot(xs, wiz_ref[...]) + biz_ref[...]
    gin = _dot(xs, win_ref[...]) + bin_ref[...]
    h = jnp.zeros_like(xs)
    for t in range(T):
        if t > 0:
            out_ref[t - 1] = _dot(h, lin4_ref[...]) + blin_ref[...]
        r = jax.nn.sigmoid(gir + _dot(h, whr_ref[...]) + bhr_ref[...])
        zg = jax.nn.sigmoid(giz + _dot(h, whz_ref[...]) + bhz_ref[...])
        n = jnp.tanh(gin + r * (_dot(h, whn_ref[...]) + bhn_ref[...]))
        h = (1.0 - zg) * n + zg * h
    out_ref[T - 1] = _dot(h, lin4_ref[...]) + blin_ref[...]


def kernel(x, edge_index, c1_W, c1_b, mean_W, mean_b, W_ih, W_hh, b_ih, b_hh,
           lin_W, lin_b):
    N, D = x.shape
    H = c1_W.shape[1]
    G = W_ih.shape[0]
    Z = lin_W.shape[0]
    T = 4
    E = edge_index.shape[1]

    NP = -(-(N + 1) // ROW_BLK) * ROW_BLK
    GRID = NP // ROW_BLK
    K = -(-E // (NW * BATCH))
    K = -(-K // NBUF) * NBUF
    EPAD = NW * K * BATCH

    f32 = jnp.float32
    pad_idx = jnp.asarray(
        N + np.arange(EPAD - E, dtype=np.int32) % (NP - N))
    src = jnp.concatenate([edge_index[0], pad_idx]).reshape(NW, K, BATCH)
    dst = jnp.concatenate([edge_index[1], pad_idx]).reshape(NW, K, BATCH)
    xp = jnp.pad(x, ((0, NP - N), (0, 0)))

    onesH = jnp.ones((BATCH, H), f32)
    zerosH = jnp.zeros((NP, H), f32)

    deg_kernel = _make_deg_kernel(NP, K, H)
    conv_kernel = _make_conv_kernel(NP, K, H)

    bd4 = lambda M: jax.scipy.linalg.block_diag(M, M, M, M)
    t4 = lambda v: jnp.tile(v, 4).reshape(1, 4 * v.shape[0])

    degp = deg_kernel(dst, onesH, zerosH)
    degp_pk = degp.reshape(N_CORES, NP // 4, 128)

    R = ROW_BLK
    PK = R // 4
    pk_spec = pl.BlockSpec((PK, 128), lambda i: (i, 0))
    part_pk_spec = pl.BlockSpec((N_CORES, PK, 128), lambda i: (0, i, 0))
    full_spec = lambda a, b: pl.BlockSpec((a, b), lambda i: (0, 0))

    xp4 = xp.reshape(NP // 4, 4 * D)

    hs1_pk, dinv_pk = pl.pallas_call(
        _tc1_body,
        grid=(GRID,),
        in_specs=[pl.BlockSpec((PK, 4 * D), lambda i: (i, 0)),
                  full_spec(4 * D, 128), part_pk_spec],
        out_specs=[pk_spec, pk_spec],
        out_shape=[jax.ShapeDtypeStruct((NP // 4, 128), f32),
                   jax.ShapeDtypeStruct((NP // 4, 128), f32)],
    )(xp4, bd4(c1_W), degp_pk)

    p = conv_kernel(hs1_pk.reshape(NP, H), src, dst, zerosH)

    hs2_pk = pl.pallas_call(
        _tc2_body,
        grid=(GRID,),
        in_specs=[pk_spec, part_pk_spec, pk_spec,
                  full_spec(1, 128), full_spec(128, 128)],
        out_specs=pk_spec,
        out_shape=jax.ShapeDtypeStruct((NP // 4, 128), f32),
    )(dinv_pk, p.reshape(N_CORES, NP // 4, 128), hs1_pk,
      t4(c1_b), bd4(mean_W))

    q = conv_kernel(hs2_pk.reshape(NP, H), src, dst, zerosH)

    Wi = W_ih.T
    Wh = W_hh.T
    gate_ws = [bd4(Wi[:, 0:H]), bd4(Wi[:, H:2 * H]), bd4(Wi[:, 2 * H:]),
               bd4(Wh[:, 0:H]), bd4(Wh[:, H:2 * H]), bd4(Wh[:, 2 * H:])]
    gate_bs = [t4(b_ih[0:H]), t4(b_ih[H:2 * H]), t4(b_ih[2 * H:]),
               t4(b_hh[0:H]), t4(b_hh[H:2 * H]), t4(b_hh[2 * H:])]

    outp = pl.pallas_call(
        functools.partial(_tc3_body, T),
        grid=(GRID,),
        in_specs=[pk_spec, part_pk_spec, pk_spec, full_spec(1, 128)]
                 + [full_spec(128, 128)] * 6 + [full_spec(128, 4 * Z)]
                 + [full_spec(1, 128)] * 6 + [full_spec(1, 4 * Z)],
        out_specs=pl.BlockSpec((T, PK, 4 * Z), lambda i: (0, i, 0)),
        out_shape=jax.ShapeDtypeStruct((T, NP // 4, 4 * Z), f32),
    )(dinv_pk, q.reshape(N_CORES, NP // 4, 128), hs2_pk, t4(mean_b),
      *gate_ws, bd4(lin_W.T), *gate_bs, t4(lin_b))

    return outp.reshape(T, NP, Z)[:, :N, :]


# speedup vs baseline: 53.8364x; 1.1622x over previous
import functools

import numpy as np
import jax
import jax.numpy as jnp
from jax import lax
from jax.experimental import pallas as pl
from jax.experimental.pallas import tpu as pltpu
from jax.experimental.pallas import tpu_sc as plsc

N_CORES = 2
N_SUB = 16
NW = N_CORES * N_SUB
BATCH = 128
ROW_BLK = 1280


def _sc_mesh():
    return plsc.VectorSubcoreMesh(core_axis_name="c", subcore_axis_name="s")


def _make_deg_kernel(NP, K, H):
    RPS = NP // N_SUB

    @functools.partial(
        pl.kernel,
        out_type=jax.ShapeDtypeStruct((N_CORES, NP, H), jnp.float32),
        mesh=_sc_mesh(),
        compiler_params=pltpu.CompilerParams(use_tc_tiling_on_sc=False),
        scratch_types=[
            pltpu.VMEM((K, BATCH), jnp.int32),
            pltpu.VMEM((BATCH, H), jnp.float32),
            pltpu.VMEM_SHARED((NP, H), jnp.float32),
        ],
    )
    def deg_kernel(dst_hbm, ones_hbm, zeros_hbm, out_hbm, dst_v, ones_v, acc):
        cid = lax.axis_index("c")
        sid = lax.axis_index("s")
        wid = cid * N_SUB + sid
        pltpu.sync_copy(dst_hbm.at[wid], dst_v)
        pltpu.sync_copy(ones_hbm, ones_v)
        pltpu.sync_copy(zeros_hbm.at[pl.ds(sid * RPS, RPS)],
                        acc.at[pl.ds(sid * RPS, RPS)])
        plsc.subcore_barrier()

        def body(j, carry):
            pltpu.sync_copy(ones_v, acc.at[dst_v.at[j]], add=True)
            return carry

        lax.fori_loop(0, K, body, 0)
        plsc.subcore_barrier()
        pltpu.sync_copy(acc.at[pl.ds(sid * RPS, RPS)],
                        out_hbm.at[cid, pl.ds(sid * RPS, RPS)])

    return deg_kernel


NBUF = 16


def _make_conv_kernel(NP, K, H):
    RPS = NP // N_SUB
    NG = K // NBUF

    @functools.partial(
        pl.kernel,
        out_type=jax.ShapeDtypeStruct((N_CORES, NP, H), jnp.float32),
        mesh=_sc_mesh(),
        compiler_params=pltpu.CompilerParams(use_tc_tiling_on_sc=False),
        scratch_types=[
            pltpu.VMEM((K, BATCH), jnp.int32),
            pltpu.VMEM((K, BATCH), jnp.int32),
            pltpu.VMEM_SHARED((NP, H), jnp.float32),
        ] + [pltpu.VMEM((BATCH, H), jnp.float32) for _ in range(NBUF)]
          + [pltpu.SemaphoreType.DMA for _ in range(NBUF)]
          + [pltpu.SemaphoreType.DMA],
    )
    def conv_kernel(table_hbm, src_hbm, dst_hbm, zeros_hbm, out_hbm,
                    src_v, dst_v, acc, *bufs_and_sems):
        rows = bufs_and_sems[:NBUF]
        gsem = bufs_and_sems[NBUF:2 * NBUF]
        ssem = bufs_and_sems[2 * NBUF]
        cid = lax.axis_index("c")
        sid = lax.axis_index("s")
        wid = cid * N_SUB + sid
        pltpu.sync_copy(src_hbm.at[wid], src_v)
        pltpu.sync_copy(dst_hbm.at[wid], dst_v)
        pltpu.sync_copy(zeros_hbm.at[pl.ds(sid * RPS, RPS)],
                        acc.at[pl.ds(sid * RPS, RPS)])
        plsc.subcore_barrier()

        for b in range(NBUF):
            pltpu.async_copy(table_hbm.at[src_v.at[b]], rows[b], gsem[b])

        def group(g, carry):
            for b in range(NBUF):
                j = g * NBUF + b
                pltpu.make_async_copy(
                    table_hbm.at[src_v.at[j]], rows[b], gsem[b]).wait()
                pltpu.async_copy(rows[b], acc.at[dst_v.at[j]], ssem,
                                 add=True)
            for b in range(NBUF):
                j = g * NBUF + b
                pltpu.make_async_copy(rows[b], acc.at[dst_v.at[j]],
                                      ssem).wait()
            jn = jnp.minimum((g + 1) * NBUF, K - NBUF)
            for b in range(NBUF):
                pltpu.async_copy(table_hbm.at[src_v.at[jn + b]], rows[b],
                                 gsem[b])
            return carry

        lax.fori_loop(0, NG, group, 0)
        for b in range(NBUF):
            pltpu.make_async_copy(
                table_hbm.at[src_v.at[K - NBUF + b]], rows[b],
                gsem[b]).wait()
        plsc.subcore_barrier()
        pltpu.sync_copy(acc.at[pl.ds(sid * RPS, RPS)],
                        out_hbm.at[cid, pl.ds(sid * RPS, RPS)])

    return conv_kernel


def _dot(a, b):
    return jnp.dot(a, b, preferred_element_type=jnp.float32)


def _tc1_body(x4_ref, w4_ref, degp_ref, hs_ref, dinv_ref):
    dinv = lax.rsqrt(1.0 + degp_ref[0] + degp_ref[1])
    h = _dot(x4_ref[...], w4_ref[...])
    dinv_ref[...] = dinv
    hs_ref[...] = h * dinv


def _tc2_body(dinv_ref, p_ref, hs1_ref, b1_ref, w4_ref, hs2_ref):
    dinv = dinv_ref[...]
    t = dinv * (p_ref[0] + p_ref[1] + hs1_ref[...]) + b1_ref[...]
    h1 = jnp.maximum(t, 0.0)
    hs2_ref[...] = _dot(h1, w4_ref[...]) * dinv


def _tc3_body(T, H, dinv_ref, q_ref, hs2_ref, b2_ref, wih_ref, whh_ref,
              linw_ref, bih_ref, bhh_ref, blin_ref, out_ref):
    dinv = dinv_ref[...]
    z = dinv * (q_ref[0] + q_ref[1] + hs2_ref[...]) + b2_ref[...]
    xsT = jnp.tanh(z).T
    for k in range(4):
        xk = xsT[32 * k:32 * k + H]
        giT = _dot(wih_ref[...], xk) + bih_ref[...]
        hT = jnp.zeros_like(xk)
        for t in range(T):
            if t > 0:
                out_ref[t - 1, :, k] = _dot(linw_ref[...], hT) + blin_ref[...]
            gcT = _dot(whh_ref[...], hT) + bhh_ref[...]
            r = jax.nn.sigmoid(giT[0:H] + gcT[0:H])
            zg = jax.nn.sigmoid(giT[H:2 * H] + gcT[H:2 * H])
            n = jnp.tanh(giT[2 * H:3 * H] + r * gcT[2 * H:3 * H])
            hT = (1.0 - zg) * n + zg * hT
        out_ref[T - 1, :, k] = _dot(linw_ref[...], hT) + blin_ref[...]


def kernel(x, edge_index, c1_W, c1_b, mean_W, mean_b, W_ih, W_hh, b_ih, b_hh,
           lin_W, lin_b):
    N, D = x.shape
    H = c1_W.shape[1]
    G = W_ih.shape[0]
    Z = lin_W.shape[0]
    T = 4
    E = edge_index.shape[1]

    NP = -(-(N + 1) // ROW_BLK) * ROW_BLK
    GRID = NP // ROW_BLK
    K = -(-E // (NW * BATCH))
    K = -(-K // NBUF) * NBUF
    EPAD = NW * K * BATCH

    f32 = jnp.float32
    PKT = NP // 4
    pad_idx = jnp.asarray(
        N + np.arange(EPAD - E, dtype=np.int32) % (NP - N))
    pos = lambda v: 4 * (v % PKT) + v // PKT
    src = pos(jnp.concatenate([edge_index[0], pad_idx])).reshape(
        NW, K, BATCH)
    dst = pos(jnp.concatenate([edge_index[1], pad_idx])).reshape(
        NW, K, BATCH)
    xp = jnp.pad(x, ((0, NP - N), (0, 0)))

    onesH = jnp.ones((BATCH, H), f32)
    zerosH = jnp.zeros((NP, H), f32)

    deg_kernel = _make_deg_kernel(NP, K, H)
    conv_kernel = _make_conv_kernel(NP, K, H)

    bd4 = lambda M: jax.scipy.linalg.block_diag(M, M, M, M)
    t4 = lambda v: jnp.tile(v, 4).reshape(1, 4 * v.shape[0])

    degp = deg_kernel(dst, onesH, zerosH)
    degp_pk = degp.reshape(N_CORES, NP // 4, 128)

    R = ROW_BLK
    PK = R // 4
    pk_spec = pl.BlockSpec((PK, 128), lambda i: (i, 0))
    part_pk_spec = pl.BlockSpec((N_CORES, PK, 128), lambda i: (0, i, 0))
    full_spec = lambda a, b: pl.BlockSpec((a, b), lambda i: (0, 0))

    xp4 = xp.reshape(4, PKT, D).transpose(1, 0, 2).reshape(PKT, 4 * D)

    hs1_pk, dinv_pk = pl.pallas_call(
        _tc1_body,
        grid=(GRID,),
        in_specs=[pl.BlockSpec((PK, 4 * D), lambda i: (i, 0)),
                  full_spec(4 * D, 128), part_pk_spec],
        out_specs=[pk_spec, pk_spec],
        out_shape=[jax.ShapeDtypeStruct((NP // 4, 128), f32),
                   jax.ShapeDtypeStruct((NP // 4, 128), f32)],
    )(xp4, bd4(c1_W), degp_pk)

    p = conv_kernel(hs1_pk.reshape(NP, H), src, dst, zerosH)

    hs2_pk = pl.pallas_call(
        _tc2_body,
        grid=(GRID,),
        in_specs=[pk_spec, part_pk_spec, pk_spec,
                  full_spec(1, 128), full_spec(128, 128)],
        out_specs=pk_spec,
        out_shape=jax.ShapeDtypeStruct((NP // 4, 128), f32),
    )(dinv_pk, p.reshape(N_CORES, NP // 4, 128), hs1_pk,
      t4(c1_b), bd4(mean_W))

    q = conv_kernel(hs2_pk.reshape(NP, H), src, dst, zerosH)

    PKB = 512
    G3 = PKT // PKB
    pk3_spec = pl.BlockSpec((PKB, 128), lambda i: (i, 0))
    outp = pl.pallas_call(
        functools.partial(_tc3_body, T, H),
        grid=(G3,),
        in_specs=[pk3_spec,
                  pl.BlockSpec((N_CORES, PKB, 128), lambda i: (0, i, 0)),
                  pk3_spec, full_spec(1, 128),
                  full_spec(G, H), full_spec(G, H), full_spec(Z, H),
                  full_spec(G, 1), full_spec(G, 1), full_spec(Z, 1)],
        out_specs=pl.BlockSpec((T, Z, 4, PKB), lambda i: (0, 0, 0, i)),
        out_shape=jax.ShapeDtypeStruct((T, Z, 4, PKT), f32),
    )(dinv_pk, q.reshape(N_CORES, NP // 4, 128), hs2_pk, t4(mean_b),
      W_ih, W_hh, lin_W,
      b_ih.reshape(G, 1), b_hh.reshape(G, 1), lin_b.reshape(Z, 1))

    return jnp.swapaxes(outp.reshape(T, Z, NP), 1, 2)[:, :N, :]


# speedup vs baseline: 53.8582x; 1.0004x over previous
import functools

import numpy as np
import jax
import jax.numpy as jnp
from jax import lax
from jax.experimental import pallas as pl
from jax.experimental.pallas import tpu as pltpu
from jax.experimental.pallas import tpu_sc as plsc

N_CORES = 2
N_SUB = 16
NW = N_CORES * N_SUB
BATCH = 128
ROW_BLK = 1280


def _sc_mesh():
    return plsc.VectorSubcoreMesh(core_axis_name="c", subcore_axis_name="s")


def _make_deg_kernel(NP, K, H):
    RPS = NP // N_SUB

    @functools.partial(
        pl.kernel,
        out_type=jax.ShapeDtypeStruct((N_CORES, NP, H), jnp.float32),
        mesh=_sc_mesh(),
        compiler_params=pltpu.CompilerParams(use_tc_tiling_on_sc=False),
        scratch_types=[
            pltpu.VMEM((K, BATCH), jnp.int32),
            pltpu.VMEM((BATCH, H), jnp.float32),
            pltpu.VMEM_SHARED((NP, H), jnp.float32),
        ],
    )
    def deg_kernel(dst_hbm, ones_hbm, zeros_hbm, out_hbm, dst_v, ones_v, acc):
        cid = lax.axis_index("c")
        sid = lax.axis_index("s")
        wid = cid * N_SUB + sid
        pltpu.sync_copy(dst_hbm.at[wid], dst_v)
        pltpu.sync_copy(ones_hbm, ones_v)
        pltpu.sync_copy(zeros_hbm.at[pl.ds(sid * RPS, RPS)],
                        acc.at[pl.ds(sid * RPS, RPS)])
        plsc.subcore_barrier()

        def body(j, carry):
            pltpu.sync_copy(ones_v, acc.at[dst_v.at[j]], add=True)
            return carry

        lax.fori_loop(0, K, body, 0)
        plsc.subcore_barrier()
        pltpu.sync_copy(acc.at[pl.ds(sid * RPS, RPS)],
                        out_hbm.at[cid, pl.ds(sid * RPS, RPS)])

    return deg_kernel


NBUF = 16


def _make_conv_kernel(NP, K, H):
    RPS = NP // N_SUB
    NG = K // NBUF

    @functools.partial(
        pl.kernel,
        out_type=jax.ShapeDtypeStruct((N_CORES, NP, H), jnp.float32),
        mesh=_sc_mesh(),
        compiler_params=pltpu.CompilerParams(use_tc_tiling_on_sc=False),
        scratch_types=[
            pltpu.VMEM((K, BATCH), jnp.int32),
            pltpu.VMEM((K, BATCH), jnp.int32),
            pltpu.VMEM_SHARED((NP, H), jnp.float32),
        ] + [pltpu.VMEM((BATCH, H), jnp.float32) for _ in range(NBUF)]
          + [pltpu.SemaphoreType.DMA for _ in range(NBUF)]
          + [pltpu.SemaphoreType.DMA],
    )
    def conv_kernel(table_hbm, src_hbm, dst_hbm, zeros_hbm, out_hbm,
                    src_v, dst_v, acc, *bufs_and_sems):
        rows = bufs_and_sems[:NBUF]
        gsem = bufs_and_sems[NBUF:2 * NBUF]
        ssem = bufs_and_sems[2 * NBUF]
        cid = lax.axis_index("c")
        sid = lax.axis_index("s")
        wid = cid * N_SUB + sid
        pltpu.sync_copy(src_hbm.at[wid], src_v)
        pltpu.sync_copy(dst_hbm.at[wid], dst_v)
        pltpu.sync_copy(zeros_hbm.at[pl.ds(sid * RPS, RPS)],
                        acc.at[pl.ds(sid * RPS, RPS)])
        plsc.subcore_barrier()

        for b in range(NBUF):
            pltpu.async_copy(table_hbm.at[src_v.at[b]], rows[b], gsem[b])

        def group(g, carry):
            for b in range(NBUF):
                j = g * NBUF + b
                pltpu.make_async_copy(
                    table_hbm.at[src_v.at[j]], rows[b], gsem[b]).wait()
                pltpu.async_copy(rows[b], acc.at[dst_v.at[j]], ssem,
                                 add=True)
            for b in range(NBUF):
                j = g * NBUF + b
                pltpu.make_async_copy(rows[b], acc.at[dst_v.at[j]],
                                      ssem).wait()
            jn = jnp.minimum((g + 1) * NBUF, K - NBUF)
            for b in range(NBUF):
                pltpu.async_copy(table_hbm.at[src_v.at[jn + b]], rows[b],
                                 gsem[b])
            return carry

        lax.fori_loop(0, NG, group, 0)
        for b in range(NBUF):
            pltpu.make_async_copy(
                table_hbm.at[src_v.at[K - NBUF + b]], rows[b],
                gsem[b]).wait()
        plsc.subcore_barrier()
        pltpu.sync_copy(acc.at[pl.ds(sid * RPS, RPS)],
                        out_hbm.at[cid, pl.ds(sid * RPS, RPS)])

    return conv_kernel


def _dot(a, b):
    return jnp.dot(a, b, preferred_element_type=jnp.float32)


def _tc1_body(x4_ref, w4_ref, degp_ref, hs_ref, dinv_ref):
    dinv = lax.rsqrt(1.0 + degp_ref[0] + degp_ref[1])
    h = _dot(x4_ref[...], w4_ref[...])
    dinv_ref[...] = dinv
    hs_ref[...] = h * dinv


def _tc2_body(dinv_ref, p_ref, hs1_ref, b1_ref, w4_ref, hs2_ref):
    dinv = dinv_ref[...]
    t = dinv * (p_ref[0] + p_ref[1] + hs1_ref[...]) + b1_ref[...]
    h1 = jnp.maximum(t, 0.0)
    hs2_ref[...] = _dot(h1, w4_ref[...]) * dinv


def _tc3_body(T, H, dinv_ref, q_ref, hs2_ref, b2_ref, wih_ref, whh_ref,
              linw_ref, bih_ref, bhh_ref, blin_ref, out_ref):
    dinv = dinv_ref[...]
    z = dinv * (q_ref[0] + q_ref[1] + hs2_ref[...]) + b2_ref[...]
    xs = jnp.tanh(z)
    for k in range(4):
        xk = xs[:, 32 * k:32 * k + H]
        giT = lax.dot_general(
            wih_ref[...], xk, (((1,), (1,)), ((), ())),
            preferred_element_type=jnp.float32) + bih_ref[...]
        hT = jnp.zeros((H, xk.shape[0]), jnp.float32)
        for t in range(T):
            if t > 0:
                out_ref[t - 1, :, k] = _dot(linw_ref[...], hT) + blin_ref[...]
            gcT = _dot(whh_ref[...], hT) + bhh_ref[...]
            r = jax.nn.sigmoid(giT[0:H] + gcT[0:H])
            zg = jax.nn.sigmoid(giT[H:2 * H] + gcT[H:2 * H])
            n = jnp.tanh(giT[2 * H:3 * H] + r * gcT[2 * H:3 * H])
            hT = (1.0 - zg) * n + zg * hT
        out_ref[T - 1, :, k] = _dot(linw_ref[...], hT) + blin_ref[...]


def kernel(x, edge_index, c1_W, c1_b, mean_W, mean_b, W_ih, W_hh, b_ih, b_hh,
           lin_W, lin_b):
    N, D = x.shape
    H = c1_W.shape[1]
    G = W_ih.shape[0]
    Z = lin_W.shape[0]
    T = 4
    E = edge_index.shape[1]

    NP = -(-(N + 1) // ROW_BLK) * ROW_BLK
    GRID = NP // ROW_BLK
    K = -(-E // (NW * BATCH))
    K = -(-K // NBUF) * NBUF
    EPAD = NW * K * BATCH

    f32 = jnp.float32
    PKT = NP // 4
    pad_idx = jnp.asarray(
        N + np.arange(EPAD - E, dtype=np.int32) % (NP - N))
    pos = lambda v: 4 * (v % PKT) + v // PKT
    src = pos(jnp.concatenate([edge_index[0], pad_idx])).reshape(
        NW, K, BATCH)
    dst = pos(jnp.concatenate([edge_index[1], pad_idx])).reshape(
        NW, K, BATCH)
    xp = jnp.pad(x, ((0, NP - N), (0, 0)))

    onesH = jnp.ones((BATCH, H), f32)
    zerosH = jnp.zeros((NP, H), f32)

    deg_kernel = _make_deg_kernel(NP, K, H)
    conv_kernel = _make_conv_kernel(NP, K, H)

    bd4 = lambda M: jax.scipy.linalg.block_diag(M, M, M, M)
    t4 = lambda v: jnp.tile(v, 4).reshape(1, 4 * v.shape[0])

    degp = deg_kernel(dst, onesH, zerosH)
    degp_pk = degp.reshape(N_CORES, NP // 4, 128)

    R = ROW_BLK
    PK = R // 4
    pk_spec = pl.BlockSpec((PK, 128), lambda i: (i, 0))
    part_pk_spec = pl.BlockSpec((N_CORES, PK, 128), lambda i: (0, i, 0))
    full_spec = lambda a, b: pl.BlockSpec((a, b), lambda i: (0, 0))

    xp4 = xp.reshape(4, PKT, D).transpose(1, 0, 2).reshape(PKT, 4 * D)

    hs1_pk, dinv_pk = pl.pallas_call(
        _tc1_body,
        grid=(GRID,),
        in_specs=[pl.BlockSpec((PK, 4 * D), lambda i: (i, 0)),
                  full_spec(4 * D, 128), part_pk_spec],
        out_specs=[pk_spec, pk_spec],
        out_shape=[jax.ShapeDtypeStruct((NP // 4, 128), f32),
                   jax.ShapeDtypeStruct((NP // 4, 128), f32)],
    )(xp4, bd4(c1_W), degp_pk)

    p = conv_kernel(hs1_pk.reshape(NP, H), src, dst, zerosH)

    hs2_pk = pl.pallas_call(
        _tc2_body,
        grid=(GRID,),
        in_specs=[pk_spec, part_pk_spec, pk_spec,
                  full_spec(1, 128), full_spec(128, 128)],
        out_specs=pk_spec,
        out_shape=jax.ShapeDtypeStruct((NP // 4, 128), f32),
    )(dinv_pk, p.reshape(N_CORES, NP // 4, 128), hs1_pk,
      t4(c1_b), bd4(mean_W))

    q = conv_kernel(hs2_pk.reshape(NP, H), src, dst, zerosH)

    PKB = 512
    G3 = PKT // PKB
    pk3_spec = pl.BlockSpec((PKB, 128), lambda i: (i, 0))
    outp = pl.pallas_call(
        functools.partial(_tc3_body, T, H),
        grid=(G3,),
        in_specs=[pk3_spec,
                  pl.BlockSpec((N_CORES, PKB, 128), lambda i: (0, i, 0)),
                  pk3_spec, full_spec(1, 128),
                  full_spec(G, H), full_spec(G, H), full_spec(Z, H),
                  full_spec(G, 1), full_spec(G, 1), full_spec(Z, 1)],
        out_specs=pl.BlockSpec((T, Z, 4, PKB), lambda i: (0, 0, 0, i)),
        out_shape=jax.ShapeDtypeStruct((T, Z, 4, PKT), f32),
    )(dinv_pk, q.reshape(N_CORES, NP // 4, 128), hs2_pk, t4(mean_b),
      W_ih, W_hh, lin_W,
      b_ih.reshape(G, 1), b_hh.reshape(G, 1), lin_b.reshape(Z, 1))

    return jnp.swapaxes(outp.reshape(T, Z, NP), 1, 2)[:, :N, :]


# speedup vs baseline: 57.2346x; 1.0627x over previous
import functools

import numpy as np
import jax
import jax.numpy as jnp
from jax import lax
from jax.experimental import pallas as pl
from jax.experimental.pallas import tpu as pltpu
from jax.experimental.pallas import tpu_sc as plsc

N_CORES = 2
N_SUB = 16
NW = N_CORES * N_SUB
BATCH = 128
ROW_BLK = 1280


def _sc_mesh():
    return plsc.VectorSubcoreMesh(core_axis_name="c", subcore_axis_name="s")


def _make_deg_kernel(NP, K, H):
    RPS = NP // N_SUB

    @functools.partial(
        pl.kernel,
        out_type=jax.ShapeDtypeStruct((N_CORES, NP, H), jnp.float32),
        mesh=_sc_mesh(),
        compiler_params=pltpu.CompilerParams(use_tc_tiling_on_sc=False),
        scratch_types=[
            pltpu.VMEM((K, BATCH), jnp.int32),
            pltpu.VMEM((BATCH, H), jnp.float32),
            pltpu.VMEM_SHARED((NP, H), jnp.float32),
        ],
    )
    def deg_kernel(dst_hbm, ones_hbm, zeros_hbm, out_hbm, dst_v, ones_v, acc):
        cid = lax.axis_index("c")
        sid = lax.axis_index("s")
        wid = cid * N_SUB + sid
        pltpu.sync_copy(dst_hbm.at[wid], dst_v)
        pltpu.sync_copy(ones_hbm, ones_v)
        pltpu.sync_copy(zeros_hbm.at[pl.ds(sid * RPS, RPS)],
                        acc.at[pl.ds(sid * RPS, RPS)])
        plsc.subcore_barrier()

        def body(j, carry):
            pltpu.sync_copy(ones_v, acc.at[dst_v.at[j]], add=True)
            return carry

        lax.fori_loop(0, K, body, 0)
        plsc.subcore_barrier()
        pltpu.sync_copy(acc.at[pl.ds(sid * RPS, RPS)],
                        out_hbm.at[cid, pl.ds(sid * RPS, RPS)])

    return deg_kernel


NBUF = 8


def _make_conv_kernel(NP, K, H):
    RPS = NP // N_SUB
    NG = K // NBUF

    @functools.partial(
        pl.kernel,
        out_type=jax.ShapeDtypeStruct((N_CORES, NP, H), jnp.float32),
        mesh=_sc_mesh(),
        compiler_params=pltpu.CompilerParams(use_tc_tiling_on_sc=False),
        scratch_types=[
            pltpu.VMEM((K, BATCH), jnp.int32),
            pltpu.VMEM((K, BATCH), jnp.int32),
            pltpu.VMEM_SHARED((NP, H), jnp.float32),
        ] + [pltpu.VMEM((BATCH, H), jnp.float32) for _ in range(2 * NBUF)]
          + [pltpu.SemaphoreType.DMA for _ in range(2 * NBUF)]
          + [pltpu.SemaphoreType.DMA, pltpu.SemaphoreType.DMA],
    )
    def conv_kernel(table_hbm, src_hbm, dst_hbm, zeros_hbm, out_hbm,
                    src_v, dst_v, acc, *bufs_and_sems):
        rows = (bufs_and_sems[:NBUF], bufs_and_sems[NBUF:2 * NBUF])
        gsem = (bufs_and_sems[2 * NBUF:3 * NBUF],
                bufs_and_sems[3 * NBUF:4 * NBUF])
        ssem = bufs_and_sems[4 * NBUF:4 * NBUF + 2]
        cid = lax.axis_index("c")
        sid = lax.axis_index("s")
        wid = cid * N_SUB + sid
        pltpu.sync_copy(src_hbm.at[wid], src_v)
        pltpu.sync_copy(dst_hbm.at[wid], dst_v)
        pltpu.sync_copy(zeros_hbm.at[pl.ds(sid * RPS, RPS)],
                        acc.at[pl.ds(sid * RPS, RPS)])
        plsc.subcore_barrier()

        for h in (0, 1):
            for b in range(NBUF):
                pltpu.async_copy(table_hbm.at[src_v.at[h * NBUF + b]],
                                 rows[h][b], gsem[h][b])

        def pair(u, carry):
            for h in (0, 1):
                g = 2 * u + h
                for b in range(NBUF):
                    j = g * NBUF + b
                    pltpu.make_async_copy(
                        table_hbm.at[src_v.at[j]], rows[h][b],
                        gsem[h][b]).wait()
                    pltpu.async_copy(rows[h][b], acc.at[dst_v.at[j]],
                                     ssem[h], add=True)
                for b in range(NBUF):
                    pltpu.make_async_copy(
                        rows[h][b], acc.at[dst_v.at[g * NBUF + b]],
                        ssem[h]).wait()
                jn = jnp.minimum((g + 2) * NBUF, K - NBUF)
                for b in range(NBUF):
                    pltpu.async_copy(table_hbm.at[src_v.at[jn + b]],
                                     rows[h][b], gsem[h][b])
            return carry

        lax.fori_loop(0, NG // 2, pair, 0)
        for h in (0, 1):
            for b in range(NBUF):
                pltpu.make_async_copy(
                    table_hbm.at[src_v.at[K - NBUF + b]], rows[h][b],
                    gsem[h][b]).wait()
        plsc.subcore_barrier()
        pltpu.sync_copy(acc.at[pl.ds(sid * RPS, RPS)],
                        out_hbm.at[cid, pl.ds(sid * RPS, RPS)])

    return conv_kernel


def _dot(a, b):
    return jnp.dot(a, b, preferred_element_type=jnp.float32)


def _tc1_body(x4_ref, w4_ref, degp_ref, hs_ref, dinv_ref):
    dinv = lax.rsqrt(1.0 + degp_ref[0] + degp_ref[1])
    h = _dot(x4_ref[...], w4_ref[...])
    dinv_ref[...] = dinv
    hs_ref[...] = h * dinv


def _tc2_body(dinv_ref, p_ref, hs1_ref, b1_ref, w4_ref, hs2_ref):
    dinv = dinv_ref[...]
    t = dinv * (p_ref[0] + p_ref[1] + hs1_ref[...]) + b1_ref[...]
    h1 = jnp.maximum(t, 0.0)
    hs2_ref[...] = _dot(h1, w4_ref[...]) * dinv


def _tc3_body(T, H, dinv_ref, q_ref, hs2_ref, b2_ref, wih_ref, whh_ref,
              linw_ref, bih_ref, bhh_ref, blin_ref, out_ref):
    dinv = dinv_ref[...]
    z = dinv * (q_ref[0] + q_ref[1] + hs2_ref[...]) + b2_ref[...]
    xs = jnp.tanh(z)
    for k in range(4):
        xk = xs[:, 32 * k:32 * k + H]
        giT = lax.dot_general(
            wih_ref[...], xk, (((1,), (1,)), ((), ())),
            preferred_element_type=jnp.float32) + bih_ref[...]
        hT = jnp.zeros((H, xk.shape[0]), jnp.float32)
        for t in range(T):
            if t > 0:
                out_ref[t - 1, :, k] = _dot(linw_ref[...], hT) + blin_ref[...]
            gcT = _dot(whh_ref[...], hT) + bhh_ref[...]
            r = jax.nn.sigmoid(giT[0:H] + gcT[0:H])
            zg = jax.nn.sigmoid(giT[H:2 * H] + gcT[H:2 * H])
            n = jnp.tanh(giT[2 * H:3 * H] + r * gcT[2 * H:3 * H])
            hT = (1.0 - zg) * n + zg * hT
        out_ref[T - 1, :, k] = _dot(linw_ref[...], hT) + blin_ref[...]


def kernel(x, edge_index, c1_W, c1_b, mean_W, mean_b, W_ih, W_hh, b_ih, b_hh,
           lin_W, lin_b):
    N, D = x.shape
    H = c1_W.shape[1]
    G = W_ih.shape[0]
    Z = lin_W.shape[0]
    T = 4
    E = edge_index.shape[1]

    NP = -(-(N + 1) // ROW_BLK) * ROW_BLK
    GRID = NP // ROW_BLK
    K = -(-E // (NW * BATCH))
    K = -(-K // (2 * NBUF)) * (2 * NBUF)
    EPAD = NW * K * BATCH

    f32 = jnp.float32
    PKT = NP // 4
    pad_idx = jnp.asarray(
        N + np.arange(EPAD - E, dtype=np.int32) % (NP - N))
    pos = lambda v: 4 * (v % PKT) + v // PKT
    src = pos(jnp.concatenate([edge_index[0], pad_idx])).reshape(
        NW, K, BATCH)
    dst = pos(jnp.concatenate([edge_index[1], pad_idx])).reshape(
        NW, K, BATCH)
    xp = jnp.pad(x, ((0, NP - N), (0, 0)))

    onesH = jnp.ones((BATCH, H), f32)
    zerosH = jnp.zeros((NP, H), f32)

    deg_kernel = _make_deg_kernel(NP, K, H)
    conv_kernel = _make_conv_kernel(NP, K, H)

    bd4 = lambda M: jax.scipy.linalg.block_diag(M, M, M, M)
    t4 = lambda v: jnp.tile(v, 4).reshape(1, 4 * v.shape[0])

    degp = deg_kernel(dst, onesH, zerosH)
    degp_pk = degp.reshape(N_CORES, NP // 4, 128)

    R = ROW_BLK
    PK = R // 4
    pk_spec = pl.BlockSpec((PK, 128), lambda i: (i, 0))
    part_pk_spec = pl.BlockSpec((N_CORES, PK, 128), lambda i: (0, i, 0))
    full_spec = lambda a, b: pl.BlockSpec((a, b), lambda i: (0, 0))

    xp4 = xp.reshape(4, PKT, D).transpose(1, 0, 2).reshape(PKT, 4 * D)

    hs1_pk, dinv_pk = pl.pallas_call(
        _tc1_body,
        grid=(GRID,),
        in_specs=[pl.BlockSpec((PK, 4 * D), lambda i: (i, 0)),
                  full_spec(4 * D, 128), part_pk_spec],
        out_specs=[pk_spec, pk_spec],
        out_shape=[jax.ShapeDtypeStruct((NP // 4, 128), f32),
                   jax.ShapeDtypeStruct((NP // 4, 128), f32)],
    )(xp4, bd4(c1_W), degp_pk)

    p = conv_kernel(hs1_pk.reshape(NP, H), src, dst, zerosH)

    hs2_pk = pl.pallas_call(
        _tc2_body,
        grid=(GRID,),
        in_specs=[pk_spec, part_pk_spec, pk_spec,
                  full_spec(1, 128), full_spec(128, 128)],
        out_specs=pk_spec,
        out_shape=jax.ShapeDtypeStruct((NP // 4, 128), f32),
    )(dinv_pk, p.reshape(N_CORES, NP // 4, 128), hs1_pk,
      t4(c1_b), bd4(mean_W))

    q = conv_kernel(hs2_pk.reshape(NP, H), src, dst, zerosH)

    PKB = 512
    G3 = PKT // PKB
    pk3_spec = pl.BlockSpec((PKB, 128), lambda i: (i, 0))
    outp = pl.pallas_call(
        functools.partial(_tc3_body, T, H),
        grid=(G3,),
        in_specs=[pk3_spec,
                  pl.BlockSpec((N_CORES, PKB, 128), lambda i: (0, i, 0)),
                  pk3_spec, full_spec(1, 128),
                  full_spec(G, H), full_spec(G, H), full_spec(Z, H),
                  full_spec(G, 1), full_spec(G, 1), full_spec(Z, 1)],
        out_specs=pl.BlockSpec((T, Z, 4, PKB), lambda i: (0, 0, 0, i)),
        out_shape=jax.ShapeDtypeStruct((T, Z, 4, PKT), f32),
    )(dinv_pk, q.reshape(N_CORES, NP // 4, 128), hs2_pk, t4(mean_b),
      W_ih, W_hh, lin_W,
      b_ih.reshape(G, 1), b_hh.reshape(G, 1), lin_b.reshape(Z, 1))

    return jnp.swapaxes(outp.reshape(T, Z, NP), 1, 2)[:, :N, :]


# speedup vs baseline: 57.5416x; 1.0054x over previous
import functools

import numpy as np
import jax
import jax.numpy as jnp
from jax import lax
from jax.experimental import pallas as pl
from jax.experimental.pallas import tpu as pltpu
from jax.experimental.pallas import tpu_sc as plsc

N_CORES = 2
N_SUB = 16
NW = N_CORES * N_SUB
BATCH = 128
ROW_BLK = 1280


def _sc_mesh():
    return plsc.VectorSubcoreMesh(core_axis_name="c", subcore_axis_name="s")


def _make_deg_kernel(NP, K, H):
    RPS = NP // N_SUB

    @functools.partial(
        pl.kernel,
        out_type=jax.ShapeDtypeStruct((N_CORES, NP, H), jnp.float32),
        mesh=_sc_mesh(),
        compiler_params=pltpu.CompilerParams(use_tc_tiling_on_sc=False),
        scratch_types=[
            pltpu.VMEM((K, BATCH), jnp.int32),
            pltpu.VMEM((BATCH, H), jnp.float32),
            pltpu.VMEM_SHARED((NP, H), jnp.float32),
        ],
    )
    def deg_kernel(dst_hbm, ones_hbm, zeros_hbm, out_hbm, dst_v, ones_v, acc):
        cid = lax.axis_index("c")
        sid = lax.axis_index("s")
        wid = cid * N_SUB + sid
        pltpu.sync_copy(dst_hbm.at[wid], dst_v)
        pltpu.sync_copy(ones_hbm, ones_v)
        pltpu.sync_copy(zeros_hbm.at[pl.ds(sid * RPS, RPS)],
                        acc.at[pl.ds(sid * RPS, RPS)])
        plsc.subcore_barrier()

        def body(j, carry):
            pltpu.sync_copy(ones_v, acc.at[dst_v.at[j]], add=True)
            return carry

        lax.fori_loop(0, K, body, 0)
        plsc.subcore_barrier()
        pltpu.sync_copy(acc.at[pl.ds(sid * RPS, RPS)],
                        out_hbm.at[cid, pl.ds(sid * RPS, RPS)])

    return deg_kernel


NBUF = 4


def _make_conv_kernel(NP, K, H):
    RPS = NP // N_SUB
    NG = K // NBUF

    @functools.partial(
        pl.kernel,
        out_type=jax.ShapeDtypeStruct((N_CORES, NP, H), jnp.float32),
        mesh=_sc_mesh(),
        compiler_params=pltpu.CompilerParams(use_tc_tiling_on_sc=False),
        scratch_types=[
            pltpu.VMEM((K, BATCH), jnp.int32),
            pltpu.VMEM((K, BATCH), jnp.int32),
            pltpu.VMEM_SHARED((NP, H), jnp.float32),
        ] + [pltpu.VMEM((BATCH, H), jnp.float32) for _ in range(2 * NBUF)]
          + [pltpu.SemaphoreType.DMA for _ in range(2 * NBUF)]
          + [pltpu.SemaphoreType.DMA, pltpu.SemaphoreType.DMA],
    )
    def conv_kernel(table_hbm, src_hbm, dst_hbm, zeros_hbm, out_hbm,
                    src_v, dst_v, acc, *bufs_and_sems):
        rows = (bufs_and_sems[:NBUF], bufs_and_sems[NBUF:2 * NBUF])
        gsem = (bufs_and_sems[2 * NBUF:3 * NBUF],
                bufs_and_sems[3 * NBUF:4 * NBUF])
        ssem = bufs_and_sems[4 * NBUF:4 * NBUF + 2]
        cid = lax.axis_index("c")
        sid = lax.axis_index("s")
        wid = cid * N_SUB + sid
        pltpu.sync_copy(src_hbm.at[wid], src_v)
        pltpu.sync_copy(dst_hbm.at[wid], dst_v)
        pltpu.sync_copy(zeros_hbm.at[pl.ds(sid * RPS, RPS)],
                        acc.at[pl.ds(sid * RPS, RPS)])
        plsc.subcore_barrier()

        for h in (0, 1):
            for b in range(NBUF):
                pltpu.async_copy(table_hbm.at[src_v.at[h * NBUF + b]],
                                 rows[h][b], gsem[h][b])

        def pair(u, carry):
            for h in (0, 1):
                g = 2 * u + h
                for b in range(NBUF):
                    j = g * NBUF + b
                    pltpu.make_async_copy(
                        table_hbm.at[src_v.at[j]], rows[h][b],
                        gsem[h][b]).wait()
                    pltpu.async_copy(rows[h][b], acc.at[dst_v.at[j]],
                                     ssem[h], add=True)
                for b in range(NBUF):
                    pltpu.make_async_copy(
                        rows[h][b], acc.at[dst_v.at[g * NBUF + b]],
                        ssem[h]).wait()
                jn = jnp.minimum((g + 2) * NBUF, K - NBUF)
                for b in range(NBUF):
                    pltpu.async_copy(table_hbm.at[src_v.at[jn + b]],
                                     rows[h][b], gsem[h][b])
            return carry

        lax.fori_loop(0, NG // 2, pair, 0)
        for h in (0, 1):
            for b in range(NBUF):
                pltpu.make_async_copy(
                    table_hbm.at[src_v.at[K - NBUF + b]], rows[h][b],
                    gsem[h][b]).wait()
        plsc.subcore_barrier()
        pltpu.sync_copy(acc.at[pl.ds(sid * RPS, RPS)],
                        out_hbm.at[cid, pl.ds(sid * RPS, RPS)])

    return conv_kernel


def _dot(a, b):
    return jnp.dot(a, b, preferred_element_type=jnp.float32)


def _tc1_body(x4_ref, w4_ref, degp_ref, hs_ref, dinv_ref):
    dinv = lax.rsqrt(1.0 + degp_ref[0] + degp_ref[1])
    h = _dot(x4_ref[...], w4_ref[...])
    dinv_ref[...] = dinv
    hs_ref[...] = h * dinv


def _tc2_body(dinv_ref, p_ref, hs1_ref, b1_ref, w4_ref, hs2_ref):
    dinv = dinv_ref[...]
    t = dinv * (p_ref[0] + p_ref[1] + hs1_ref[...]) + b1_ref[...]
    h1 = jnp.maximum(t, 0.0)
    hs2_ref[...] = _dot(h1, w4_ref[...]) * dinv


def _tc3_body(T, H, dinv_ref, q_ref, hs2_ref, b2_ref, wih_ref, whh_ref,
              linw_ref, bih_ref, bhh_ref, blin_ref, out_ref):
    dinv = dinv_ref[...]
    z = dinv * (q_ref[0] + q_ref[1] + hs2_ref[...]) + b2_ref[...]
    xs = jnp.tanh(z)
    for k in range(4):
        xk = xs[:, 32 * k:32 * k + H]
        giT = lax.dot_general(
            wih_ref[...], xk, (((1,), (1,)), ((), ())),
            preferred_element_type=jnp.float32) + bih_ref[...]
        hT = jnp.zeros((H, xk.shape[0]), jnp.float32)
        for t in range(T):
            if t > 0:
                out_ref[t - 1, :, k] = _dot(linw_ref[...], hT) + blin_ref[...]
            gcT = _dot(whh_ref[...], hT) + bhh_ref[...]
            r = jax.nn.sigmoid(giT[0:H] + gcT[0:H])
            zg = jax.nn.sigmoid(giT[H:2 * H] + gcT[H:2 * H])
            n = jnp.tanh(giT[2 * H:3 * H] + r * gcT[2 * H:3 * H])
            hT = (1.0 - zg) * n + zg * hT
        out_ref[T - 1, :, k] = _dot(linw_ref[...], hT) + blin_ref[...]


def kernel(x, edge_index, c1_W, c1_b, mean_W, mean_b, W_ih, W_hh, b_ih, b_hh,
           lin_W, lin_b):
    N, D = x.shape
    H = c1_W.shape[1]
    G = W_ih.shape[0]
    Z = lin_W.shape[0]
    T = 4
    E = edge_index.shape[1]

    NP = -(-(N + 1) // ROW_BLK) * ROW_BLK
    GRID = NP // ROW_BLK
    K = -(-E // (NW * BATCH))
    K = -(-K // (2 * NBUF)) * (2 * NBUF)
    EPAD = NW * K * BATCH

    f32 = jnp.float32
    PKT = NP // 4
    pad_idx = jnp.asarray(
        N + np.arange(EPAD - E, dtype=np.int32) % (NP - N))
    pos = lambda v: 4 * (v % PKT) + v // PKT
    src = pos(jnp.concatenate([edge_index[0], pad_idx])).reshape(
        NW, K, BATCH)
    dst = pos(jnp.concatenate([edge_index[1], pad_idx])).reshape(
        NW, K, BATCH)
    xp = jnp.pad(x, ((0, NP - N), (0, 0)))

    onesH = jnp.ones((BATCH, H), f32)
    zerosH = jnp.zeros((NP, H), f32)

    deg_kernel = _make_deg_kernel(NP, K, H)
    conv_kernel = _make_conv_kernel(NP, K, H)

    bd4 = lambda M: jax.scipy.linalg.block_diag(M, M, M, M)
    t4 = lambda v: jnp.tile(v, 4).reshape(1, 4 * v.shape[0])

    degp = deg_kernel(dst, onesH, zerosH)
    degp_pk = degp.reshape(N_CORES, NP // 4, 128)

    R = ROW_BLK
    PK = R // 4
    pk_spec = pl.BlockSpec((PK, 128), lambda i: (i, 0))
    part_pk_spec = pl.BlockSpec((N_CORES, PK, 128), lambda i: (0, i, 0))
    full_spec = lambda a, b: pl.BlockSpec((a, b), lambda i: (0, 0))

    xp4 = xp.reshape(4, PKT, D).transpose(1, 0, 2).reshape(PKT, 4 * D)

    hs1_pk, dinv_pk = pl.pallas_call(
        _tc1_body,
        grid=(GRID,),
        in_specs=[pl.BlockSpec((PK, 4 * D), lambda i: (i, 0)),
                  full_spec(4 * D, 128), part_pk_spec],
        out_specs=[pk_spec, pk_spec],
        out_shape=[jax.ShapeDtypeStruct((NP // 4, 128), f32),
                   jax.ShapeDtypeStruct((NP // 4, 128), f32)],
    )(xp4, bd4(c1_W), degp_pk)

    p = conv_kernel(hs1_pk.reshape(NP, H), src, dst, zerosH)

    hs2_pk = pl.pallas_call(
        _tc2_body,
        grid=(GRID,),
        in_specs=[pk_spec, part_pk_spec, pk_spec,
                  full_spec(1, 128), full_spec(128, 128)],
        out_specs=pk_spec,
        out_shape=jax.ShapeDtypeStruct((NP // 4, 128), f32),
    )(dinv_pk, p.reshape(N_CORES, NP // 4, 128), hs1_pk,
      t4(c1_b), bd4(mean_W))

    q = conv_kernel(hs2_pk.reshape(NP, H), src, dst, zerosH)

    PKB = 512
    G3 = PKT // PKB
    pk3_spec = pl.BlockSpec((PKB, 128), lambda i: (i, 0))
    outp = pl.pallas_call(
        functools.partial(_tc3_body, T, H),
        grid=(G3,),
        in_specs=[pk3_spec,
                  pl.BlockSpec((N_CORES, PKB, 128), lambda i: (0, i, 0)),
                  pk3_spec, full_spec(1, 128),
                  full_spec(G, H), full_spec(G, H), full_spec(Z, H),
                  full_spec(G, 1), full_spec(G, 1), full_spec(Z, 1)],
        out_specs=pl.BlockSpec((T, Z, 4, PKB), lambda i: (0, 0, 0, i)),
        out_shape=jax.ShapeDtypeStruct((T, Z, 4, PKT), f32),
    )(dinv_pk, q.reshape(N_CORES, NP // 4, 128), hs2_pk, t4(mean_b),
      W_ih, W_hh, lin_W,
      b_ih.reshape(G, 1), b_hh.reshape(G, 1), lin_b.reshape(Z, 1))

    return jnp.swapaxes(outp.reshape(T, Z, NP), 1, 2)[:, :N, :]


# speedup vs baseline: 58.2887x; 1.0130x over previous
import functools

import numpy as np
import jax
import jax.numpy as jnp
from jax import lax
from jax.experimental import pallas as pl
from jax.experimental.pallas import tpu as pltpu
from jax.experimental.pallas import tpu_sc as plsc

N_CORES = 2
N_SUB = 16
NW = N_CORES * N_SUB
BATCH = 128
ROW_BLK = 1280


def _sc_mesh():
    return plsc.VectorSubcoreMesh(core_axis_name="c", subcore_axis_name="s")


def _make_deg_kernel(NP, K, H):
    RPS = NP // N_SUB

    @functools.partial(
        pl.kernel,
        out_type=jax.ShapeDtypeStruct((N_CORES, NP, H), jnp.float32),
        mesh=_sc_mesh(),
        compiler_params=pltpu.CompilerParams(use_tc_tiling_on_sc=False),
        scratch_types=[
            pltpu.VMEM((K, BATCH), jnp.int32),
            pltpu.VMEM((BATCH, H), jnp.float32),
            pltpu.VMEM_SHARED((NP, H), jnp.float32),
        ],
    )
    def deg_kernel(dst_hbm, ones_hbm, zeros_hbm, out_hbm, dst_v, ones_v, acc):
        cid = lax.axis_index("c")
        sid = lax.axis_index("s")
        wid = cid * N_SUB + sid
        pltpu.sync_copy(dst_hbm.at[wid], dst_v)
        pltpu.sync_copy(ones_hbm, ones_v)
        pltpu.sync_copy(zeros_hbm.at[pl.ds(sid * RPS, RPS)],
                        acc.at[pl.ds(sid * RPS, RPS)])
        plsc.subcore_barrier()

        def body(j, carry):
            pltpu.sync_copy(ones_v, acc.at[dst_v.at[j]], add=True)
            return carry

        lax.fori_loop(0, K, body, 0)
        plsc.subcore_barrier()
        pltpu.sync_copy(acc.at[pl.ds(sid * RPS, RPS)],
                        out_hbm.at[cid, pl.ds(sid * RPS, RPS)])

    return deg_kernel


NBUF = 4


def _make_conv_kernel(NP, K, H):
    RPS = NP // N_SUB
    NG = K // NBUF

    @functools.partial(
        pl.kernel,
        out_type=jax.ShapeDtypeStruct((N_CORES, NP, H), jnp.float32),
        mesh=_sc_mesh(),
        compiler_params=pltpu.CompilerParams(use_tc_tiling_on_sc=False),
        scratch_types=[
            pltpu.VMEM((K, BATCH), jnp.int32),
            pltpu.VMEM((K, BATCH), jnp.int32),
            pltpu.VMEM_SHARED((NP, H), jnp.float32),
        ] + [pltpu.VMEM((BATCH, H), jnp.float32) for _ in range(2 * NBUF)]
          + [pltpu.SemaphoreType.DMA for _ in range(2 * NBUF)]
          + [pltpu.SemaphoreType.DMA, pltpu.SemaphoreType.DMA],
    )
    def conv_kernel(table_hbm, src_hbm, dst_hbm, zeros_hbm, out_hbm,
                    src_v, dst_v, acc, *bufs_and_sems):
        rows = (bufs_and_sems[:NBUF], bufs_and_sems[NBUF:2 * NBUF])
        gsem = (bufs_and_sems[2 * NBUF:3 * NBUF],
                bufs_and_sems[3 * NBUF:4 * NBUF])
        ssem = bufs_and_sems[4 * NBUF:4 * NBUF + 2]
        cid = lax.axis_index("c")
        sid = lax.axis_index("s")
        wid = cid * N_SUB + sid
        pltpu.sync_copy(src_hbm.at[wid], src_v)
        pltpu.sync_copy(dst_hbm.at[wid], dst_v)
        pltpu.sync_copy(zeros_hbm.at[pl.ds(sid * RPS, RPS)],
                        acc.at[pl.ds(sid * RPS, RPS)])
        plsc.subcore_barrier()

        for h in (0, 1):
            for b in range(NBUF):
                pltpu.async_copy(table_hbm.at[src_v.at[h * NBUF + b]],
                                 rows[h][b], gsem[h][b])

        def pair(u, carry):
            for h in (0, 1):
                g = 2 * u + h
                for b in range(NBUF):
                    j = g * NBUF + b
                    pltpu.make_async_copy(
                        table_hbm.at[src_v.at[j]], rows[h][b],
                        gsem[h][b]).wait()
                    pltpu.async_copy(rows[h][b], acc.at[dst_v.at[j]],
                                     ssem[h], add=True)
                for b in range(NBUF):
                    pltpu.make_async_copy(
                        rows[h][b], acc.at[dst_v.at[g * NBUF + b]],
                        ssem[h]).wait()
                jn = jnp.minimum((g + 2) * NBUF, K - NBUF)
                for b in range(NBUF):
                    pltpu.async_copy(table_hbm.at[src_v.at[jn + b]],
                                     rows[h][b], gsem[h][b])
            return carry

        lax.fori_loop(0, NG // 2, pair, 0)
        for h in (0, 1):
            for b in range(NBUF):
                pltpu.make_async_copy(
                    table_hbm.at[src_v.at[K - NBUF + b]], rows[h][b],
                    gsem[h][b]).wait()
        plsc.subcore_barrier()
        pltpu.sync_copy(acc.at[pl.ds(sid * RPS, RPS)],
                        out_hbm.at[cid, pl.ds(sid * RPS, RPS)])

    return conv_kernel


def _dot(a, b):
    return jnp.dot(a, b, preferred_element_type=jnp.float32)


def _tc1_body(x4_ref, w4_ref, degp_ref, hs_ref, dinv_ref):
    dinv = lax.rsqrt(1.0 + degp_ref[0] + degp_ref[1])
    h = _dot(x4_ref[...], w4_ref[...])
    dinv_ref[...] = dinv
    hs_ref[...] = h * dinv


def _tc2_body(dinv_ref, p_ref, hs1_ref, b1_ref, w4_ref, hs2_ref):
    dinv = dinv_ref[...]
    t = dinv * (p_ref[0] + p_ref[1] + hs1_ref[...]) + b1_ref[...]
    h1 = jnp.maximum(t, 0.0)
    hs2_ref[...] = _dot(h1, w4_ref[...]) * dinv


def _tc3_body(T, H, dinv_ref, q_ref, hs2_ref, b2_ref, wih_ref, whh_ref,
              linw_ref, bih_ref, bhh_ref, blin_ref, out_ref):
    dinv = dinv_ref[...]
    z = dinv * (q_ref[0] + q_ref[1] + hs2_ref[...]) + b2_ref[...]
    xs = jnp.tanh(z)
    for k in range(4):
        xk = xs[:, 32 * k:32 * k + H]
        giT = lax.dot_general(
            wih_ref[...], xk, (((1,), (1,)), ((), ())),
            preferred_element_type=jnp.float32) + bih_ref[...]
        hT = jnp.zeros((H, xk.shape[0]), jnp.float32)
        for t in range(T):
            if t > 0:
                out_ref[t - 1, :, k] = _dot(linw_ref[...], hT) + blin_ref[...]
            gcT = _dot(whh_ref[...], hT) + bhh_ref[...]
            r = jax.nn.sigmoid(giT[0:H] + gcT[0:H])
            zg = jax.nn.sigmoid(giT[H:2 * H] + gcT[H:2 * H])
            n = jnp.tanh(giT[2 * H:3 * H] + r * gcT[2 * H:3 * H])
            hT = (1.0 - zg) * n + zg * hT
        out_ref[T - 1, :, k] = _dot(linw_ref[...], hT) + blin_ref[...]


def kernel(x, edge_index, c1_W, c1_b, mean_W, mean_b, W_ih, W_hh, b_ih, b_hh,
           lin_W, lin_b):
    N, D = x.shape
    H = c1_W.shape[1]
    G = W_ih.shape[0]
    Z = lin_W.shape[0]
    T = 4
    E = edge_index.shape[1]

    NP = -(-(N + 1) // ROW_BLK) * ROW_BLK
    GRID = NP // ROW_BLK
    K = -(-E // (NW * BATCH))
    K = -(-K // (2 * NBUF)) * (2 * NBUF)
    EPAD = NW * K * BATCH

    f32 = jnp.float32
    PKT = NP // 4
    pad_v = N + np.arange(EPAD - E, dtype=np.int64) % (NP - N)
    pad_pos = jnp.asarray(
        (4 * (pad_v % PKT) + pad_v // PKT).astype(np.int32))

    def pos(v):
        k = ((v >= PKT).astype(jnp.int32) + (v >= 2 * PKT)
             + (v >= 3 * PKT))
        return 4 * v - (4 * PKT - 1) * k

    src = jnp.concatenate([pos(edge_index[0]), pad_pos]).reshape(
        NW, K, BATCH)
    dst = jnp.concatenate([pos(edge_index[1]), pad_pos]).reshape(
        NW, K, BATCH)
    xp = jnp.pad(x, ((0, NP - N), (0, 0)))

    onesH = jnp.ones((BATCH, H), f32)
    zerosH = jnp.zeros((NP, H), f32)

    deg_kernel = _make_deg_kernel(NP, K, H)
    conv_kernel = _make_conv_kernel(NP, K, H)

    bd4 = lambda M: jax.scipy.linalg.block_diag(M, M, M, M)
    t4 = lambda v: jnp.tile(v, 4).reshape(1, 4 * v.shape[0])

    degp = deg_kernel(dst, onesH, zerosH)
    degp_pk = degp.reshape(N_CORES, NP // 4, 128)

    R = ROW_BLK
    PK = R // 4
    pk_spec = pl.BlockSpec((PK, 128), lambda i: (i, 0))
    part_pk_spec = pl.BlockSpec((N_CORES, PK, 128), lambda i: (0, i, 0))
    full_spec = lambda a, b: pl.BlockSpec((a, b), lambda i: (0, 0))

    xp4 = xp.reshape(4, PKT, D).transpose(1, 0, 2).reshape(PKT, 4 * D)

    hs1_pk, dinv_pk = pl.pallas_call(
        _tc1_body,
        grid=(GRID,),
        in_specs=[pl.BlockSpec((PK, 4 * D), lambda i: (i, 0)),
                  full_spec(4 * D, 128), part_pk_spec],
        out_specs=[pk_spec, pk_spec],
        out_shape=[jax.ShapeDtypeStruct((NP // 4, 128), f32),
                   jax.ShapeDtypeStruct((NP // 4, 128), f32)],
    )(xp4, bd4(c1_W), degp_pk)

    p = conv_kernel(hs1_pk.reshape(NP, H), src, dst, zerosH)

    hs2_pk = pl.pallas_call(
        _tc2_body,
        grid=(GRID,),
        in_specs=[pk_spec, part_pk_spec, pk_spec,
                  full_spec(1, 128), full_spec(128, 128)],
        out_specs=pk_spec,
        out_shape=jax.ShapeDtypeStruct((NP // 4, 128), f32),
    )(dinv_pk, p.reshape(N_CORES, NP // 4, 128), hs1_pk,
      t4(c1_b), bd4(mean_W))

    q = conv_kernel(hs2_pk.reshape(NP, H), src, dst, zerosH)

    PKB = 512
    G3 = PKT // PKB
    pk3_spec = pl.BlockSpec((PKB, 128), lambda i: (i, 0))
    outp = pl.pallas_call(
        functools.partial(_tc3_body, T, H),
        grid=(G3,),
        in_specs=[pk3_spec,
                  pl.BlockSpec((N_CORES, PKB, 128), lambda i: (0, i, 0)),
                  pk3_spec, full_spec(1, 128),
                  full_spec(G, H), full_spec(G, H), full_spec(Z, H),
                  full_spec(G, 1), full_spec(G, 1), full_spec(Z, 1)],
        out_specs=pl.BlockSpec((T, Z, 4, PKB), lambda i: (0, 0, 0, i)),
        out_shape=jax.ShapeDtypeStruct((T, Z, 4, PKT), f32),
    )(dinv_pk, q.reshape(N_CORES, NP // 4, 128), hs2_pk, t4(mean_b),
      W_ih, W_hh, lin_W,
      b_ih.reshape(G, 1), b_hh.reshape(G, 1), lin_b.reshape(Z, 1))

    return jnp.swapaxes(outp.reshape(T, Z, NP), 1, 2)[:, :N, :]
